# Initial kernel scaffold; baseline (speedup 1.0000x reference)
import jax
import jax.numpy as jnp
from jax.experimental import pallas as pl


def kernel(x, edge_index, edge_attr, batch, W0, We0, att0, b0, W1, We1, att1, b1, Wout, bout):
    raise NotImplementedError("write your pallas kernel here")


# speedup vs baseline: 55.3221x; 55.3221x over previous
import functools

import jax
import jax.numpy as jnp
from jax import lax
from jax.experimental import pallas as pl
from jax.experimental.pallas import tpu as pltpu
from jax.experimental.pallas import tpu_sc as plsc

_N = 10000
_E = 320000
_HC = 128
_C = 64
_DE = 16
_NG = 16
_NCLS = 4

_NC = 2
_NS = 16
_NW = _NC * _NS
_G = 64
_TG = _E // _G
_GPW = _TG // _NW
_REM = _TG - _GPW * _NW
_ZROWS = 640


def _sc_edge_body(src_h, dst_h, atbl_h, ae_h, xl_h, z_h, out_h,
                  src_v, dst_v, ae_v, atbl_v, rows_v,
                  agg_sh, sem):
    cid = lax.axis_index("c")
    sid = lax.axis_index("s")
    wid = sid * _NC + cid

    pltpu.sync_copy(z_h, agg_sh.at[pl.ds(sid * 624, _ZROWS)])
    pltpu.sync_copy(atbl_h, atbl_v)
    plsc.subcore_barrier()

    g0 = wid * _GPW + jnp.minimum(wid, _REM)
    g1 = g0 + _GPW + jnp.where(wid < _REM, 1, 0)

    def group(g, carry):
        base = pl.multiple_of(g * _G, _G)
        pltpu.sync_copy(src_h.at[pl.ds(base, _G)], src_v)
        pltpu.sync_copy(dst_h.at[pl.ds(base, _G)], dst_v)
        pltpu.sync_copy(ae_h.at[pl.ds(base * 2, 2 * _G)], ae_v)
        pltpu.async_copy(xl_h.at[src_v], rows_v, sem).wait()

        lane = lax.iota(jnp.int32, 16)
        for j in range(_G // 16):
            s16 = src_v[pl.ds(j * 16, 16)]
            d16 = dst_v[pl.ds(j * 16, 16)]
            d4 = d16 * 4
            s4 = s16 * 4
            ai0 = plsc.load_gather(atbl_v, [d4])
            ai1 = plsc.load_gather(atbl_v, [d4 + 1])
            aj0 = plsc.load_gather(atbl_v, [s4 + 2])
            aj1 = plsc.load_gather(atbl_v, [s4 + 3])
            ei = j * 32 + lane * 2
            ae0 = plsc.load_gather(ae_v, [ei])
            ae1 = plsc.load_gather(ae_v, [ei + 1])
            s0 = ai0 + aj0 + ae0
            s1 = ai1 + aj1 + ae1
            s0 = jnp.where(s0 >= 0.0, s0, s0 * 0.2)
            s1 = jnp.where(s1 >= 0.0, s1, s1 * 0.2)
            m = jnp.maximum(s0, s1)
            e0 = jnp.exp(s0 - m)
            e1 = jnp.exp(s1 - m)
            inv = 1.0 / (e0 + e1)
            a0v = e0 * inv
            a1v = e1 * inv
            for k in range(16):
                r = j * 16 + k
                a0s = a0v[k]
                a1s = a1v[k]
                for q in range(4):
                    rows_v[r, pl.ds(q * 16, 16)] = (
                        rows_v[r, pl.ds(q * 16, 16)] * a0s)
                for q in range(4, 8):
                    rows_v[r, pl.ds(q * 16, 16)] = (
                        rows_v[r, pl.ds(q * 16, 16)] * a1s)

        pltpu.sync_copy(rows_v, agg_sh.at[dst_v], add=True)
        return carry

    lax.fori_loop(g0, g1, group, 0)
    plsc.subcore_barrier()
    start = 8 * (sid * 78 + jnp.minimum(sid, 2))

    @pl.when(sid < 2)
    def _read_wide():
        pltpu.sync_copy(agg_sh.at[pl.ds(start, 632)],
                        out_h.at[pl.ds(cid * _N + start, 632)])

    @pl.when(sid >= 2)
    def _read_narrow():
        pltpu.sync_copy(agg_sh.at[pl.ds(start, 624)],
                        out_h.at[pl.ds(cid * _N + start, 624)])


@functools.cache
def _get_sc_edge():
    return pl.kernel(
        _sc_edge_body,
        out_type=jax.ShapeDtypeStruct((2 * _N, _HC), jnp.float32),
        mesh=plsc.VectorSubcoreMesh(core_axis_name="c", subcore_axis_name="s",
                                    num_cores=_NC, num_subcores=_NS),
        compiler_params=pltpu.CompilerParams(needs_layout_passes=False),
        scratch_types=[
            pltpu.VMEM((_G,), jnp.int32),
            pltpu.VMEM((_G,), jnp.int32),
            pltpu.VMEM((2 * _G,), jnp.float32),
            pltpu.VMEM((4 * _N,), jnp.float32),
            pltpu.VMEM((_G, _HC), jnp.float32),
            pltpu.VMEM_SHARED((_N, _HC), jnp.float32),
            pltpu.SemaphoreType.DMA,
        ],
    )


_BN = 2000
_BE = 4000


def _node_body(x_ref, wt_ref, am_ref, xl_ref, a_ref):
    xl = jnp.dot(x_ref[...], wt_ref[...], preferred_element_type=jnp.float32)
    xl_ref[...] = xl
    a_ref[...] = jnp.dot(xl, am_ref[...], preferred_element_type=jnp.float32)


def _tc_node(x, wt, am):
    return pl.pallas_call(
        _node_body,
        grid=(_N // _BN,),
        in_specs=[
            pl.BlockSpec((_BN, _HC), lambda i: (i, 0)),
            pl.BlockSpec((_HC, _HC), lambda i: (0, 0)),
            pl.BlockSpec((_HC, 4), lambda i: (0, 0)),
        ],
        out_specs=[
            pl.BlockSpec((_BN, _HC), lambda i: (i, 0)),
            pl.BlockSpec((_BN, 4), lambda i: (i, 0)),
        ],
        out_shape=[
            jax.ShapeDtypeStruct((_N, _HC), jnp.float32),
            jax.ShapeDtypeStruct((_N, 4), jnp.float32),
        ],
    )(x, wt, am)


def _combine_body(p_ref, b_ref, wt_ref, am_ref, xl_ref, a_ref):
    h = jnp.maximum(p_ref[0] + p_ref[1] + b_ref[...], 0.0)
    xl = jnp.dot(h, wt_ref[...], preferred_element_type=jnp.float32)
    xl_ref[...] = xl
    a_ref[...] = jnp.dot(xl, am_ref[...], preferred_element_type=jnp.float32)


def _tc_combine(parts, brow, wt, am):
    return pl.pallas_call(
        _combine_body,
        grid=(_N // _BN,),
        in_specs=[
            pl.BlockSpec((2, _BN, _HC), lambda i: (0, i, 0)),
            pl.BlockSpec((1, _HC), lambda i: (0, 0)),
            pl.BlockSpec((_HC, _HC), lambda i: (0, 0)),
            pl.BlockSpec((_HC, 4), lambda i: (0, 0)),
        ],
        out_specs=[
            pl.BlockSpec((_BN, _HC), lambda i: (i, 0)),
            pl.BlockSpec((_BN, 4), lambda i: (i, 0)),
        ],
        out_shape=[
            jax.ShapeDtypeStruct((_N, _HC), jnp.float32),
            jax.ShapeDtypeStruct((_N, 4), jnp.float32),
        ],
    )(parts, brow, wt, am)


def _ea_body(ea_ref, b_ref, o_ref):
    o_ref[...] = jnp.dot(ea_ref[...], b_ref[...],
                         preferred_element_type=jnp.float32)


def _tc_ea(ea_view, bcat):
    e8 = _E // 8
    return pl.pallas_call(
        _ea_body,
        grid=(e8 // _BE,),
        in_specs=[
            pl.BlockSpec((_BE, _HC), lambda i: (i, 0)),
            pl.BlockSpec((_HC, 32), lambda i: (0, 0)),
        ],
        out_specs=pl.BlockSpec((_BE, 32), lambda i: (i, 0)),
        out_shape=jax.ShapeDtypeStruct((e8, 32), jnp.float32),
    )(ea_view, bcat)


def _final_body(p_ref, b_ref, bt_ref, wo_ref, bo_ref, o_ref, sum_acc, cnt_acc):
    i = pl.program_id(0)

    @pl.when(i == 0)
    def _init():
        sum_acc[...] = jnp.zeros_like(sum_acc)
        cnt_acc[...] = jnp.zeros_like(cnt_acc)

    h = jnp.maximum(p_ref[0] + p_ref[1] + b_ref[...], 0.0)
    oh = (bt_ref[...] == lax.broadcasted_iota(jnp.int32, (1, _NG), 1)
          ).astype(jnp.float32)
    dnum = (((0,), (0,)), ((), ()))
    sum_acc[...] += lax.dot_general(oh, h, dnum,
                                    preferred_element_type=jnp.float32)
    cnt_acc[...] += lax.dot_general(oh, jnp.ones((_BN, _HC), jnp.float32),
                                    dnum, preferred_element_type=jnp.float32)

    @pl.when(i == _N // _BN - 1)
    def _fin():
        pooled = sum_acc[...] / jnp.maximum(cnt_acc[...], 1.0)
        logits = jnp.dot(pooled, wo_ref[...],
                         preferred_element_type=jnp.float32) + bo_ref[...]
        m = jnp.max(logits, axis=1, keepdims=True)
        sh = logits - m
        o_ref[...] = sh - jnp.log(jnp.sum(jnp.exp(sh), axis=1, keepdims=True))


def _tc_final(parts, brow, batch2d, wot, borow):
    return pl.pallas_call(
        _final_body,
        grid=(_N // _BN,),
        in_specs=[
            pl.BlockSpec((2, _BN, _HC), lambda i: (0, i, 0)),
            pl.BlockSpec((1, _HC), lambda i: (0, 0)),
            pl.BlockSpec((_BN, 1), lambda i: (i, 0)),
            pl.BlockSpec((_HC, _NCLS), lambda i: (0, 0)),
            pl.BlockSpec((1, _NCLS), lambda i: (0, 0)),
        ],
        out_specs=pl.BlockSpec((_NG, _NCLS), lambda i: (0, 0)),
        out_shape=jax.ShapeDtypeStruct((_NG, _NCLS), jnp.float32),
        scratch_shapes=[
            pltpu.VMEM((_NG, _HC), jnp.float32),
            pltpu.VMEM((_NG, _HC), jnp.float32),
        ],
    )(parts, brow, batch2d, wot, borow)


def _build_A(att):
    a = jnp.zeros((_HC, 4), jnp.float32)
    a = a.at[0:_C, 0].set(att[0, 0, 0:_C])
    a = a.at[_C:_HC, 1].set(att[0, 1, 0:_C])
    a = a.at[0:_C, 2].set(att[0, 0, _C:2 * _C])
    a = a.at[_C:_HC, 3].set(att[0, 1, _C:2 * _C])
    return a


def _build_B(we, att):
    ve = jnp.stack(
        [we[h * _C:(h + 1) * _C, :].T @ att[0, h, 2 * _C:] for h in range(2)],
        axis=1)
    return jnp.kron(jnp.eye(8, dtype=jnp.float32), ve)


def kernel(x, edge_index, edge_attr, batch, W0, We0, att0, b0,
           W1, We1, att1, b1, Wout, bout):
    src = edge_index[0].astype(jnp.int32)
    dst = edge_index[1].astype(jnp.int32)
    batch2d = batch.astype(jnp.int32).reshape(_N, 1)

    A0 = _build_A(att0)
    A1 = _build_A(att1)
    bcat = jnp.concatenate([_build_B(We0, att0), _build_B(We1, att1)], axis=1)
    ea_view = edge_attr.reshape(_E // 8, _HC)
    zrows = jnp.zeros((_ZROWS, _HC), jnp.float32)

    ae_all = _tc_ea(ea_view, bcat)
    ae0 = ae_all[:, :16].reshape(-1)
    ae1 = ae_all[:, 16:].reshape(-1)

    sc_edge = _get_sc_edge()
    xl0, a0 = _tc_node(x, W0.T, A0)
    parts0 = sc_edge(src, dst, a0.reshape(-1), ae0, xl0,
                     zrows).reshape(2, _N, _HC)
    xl1, a1 = _tc_combine(parts0, b0.reshape(1, _HC), W1.T, A1)
    parts1 = sc_edge(src, dst, a1.reshape(-1), ae1, xl1,
                     zrows).reshape(2, _N, _HC)
    return _tc_final(parts1, b1.reshape(1, _HC), batch2d,
                     Wout.T, bout.reshape(1, _NCLS))


# speedup vs baseline: 66.0204x; 1.1934x over previous
import functools

import jax
import jax.numpy as jnp
from jax import lax
from jax.experimental import pallas as pl
from jax.experimental.pallas import tpu as pltpu
from jax.experimental.pallas import tpu_sc as plsc

_N = 10000
_E = 320000
_HC = 128
_C = 64
_DE = 16
_NG = 16
_NCLS = 4

_NC = 2
_NS = 16
_NW = _NC * _NS
_G = 112
_GPT = 90
_EPT = _G * _GPT
_EPAD = _NW * _EPT
_NP = _N + 8
_ZROWS = 640


def _sc_edge_body(src_h, dst_h, tbl_h, ae_h, xl_h, z_h, out_h,
                  src_v0, dst_v0, ae_v0, rows_v0,
                  src_v1, dst_v1, ae_v1, rows_v1,
                  tbl_v, agg_sh, sem0, sem1):
    cid = lax.axis_index("c")
    sid = lax.axis_index("s")
    wid = sid * _NC + cid

    pltpu.sync_copy(z_h, agg_sh.at[pl.ds(sid * 624, _ZROWS)])
    pltpu.sync_copy(tbl_h, tbl_v)
    plsc.subcore_barrier()

    ebase = wid * _EPT
    sets = ((src_v0, dst_v0, ae_v0, rows_v0, sem0),
            (src_v1, dst_v1, ae_v1, rows_v1, sem1))

    def fetch_idx(g, s):
        base = pl.multiple_of(ebase + g * _G, 16)
        pltpu.sync_copy(src_h.at[pl.ds(base, _G)], s[0])
        pltpu.sync_copy(dst_h.at[pl.ds(base, _G)], s[1])
        pltpu.sync_copy(ae_h.at[pl.ds(base * 2, 2 * _G)], s[2])

    def issue(s):
        pltpu.async_copy(xl_h.at[s[0]], s[3], s[4])

    def wait(s):
        pltpu.make_async_copy(xl_h.at[s[0]], s[3], s[4]).wait()

    def compute(s):
        src_v, dst_v, ae_v, rows_v, _sem = s
        lane = lax.iota(jnp.int32, 16)

        def jbody(j, carry):
            sl = pl.ds(j * 16, 16)
            s16 = src_v[sl]
            d16 = dst_v[sl]
            wi = plsc.load_gather(tbl_v, [d16 * 2])
            wj = plsc.load_gather(tbl_v, [s16 * 2 + 1])
            ai0, ai1 = plsc.unpack(plsc.bitcast(wi, jnp.bfloat16),
                                   format=plsc.PackFormat.INTERLEAVED)
            aj0, aj1 = plsc.unpack(plsc.bitcast(wj, jnp.bfloat16),
                                   format=plsc.PackFormat.INTERLEAVED)
            ei = j * 32 + lane * 2
            ae0 = plsc.load_gather(ae_v, [ei])
            ae1 = plsc.load_gather(ae_v, [ei + 1])
            s0 = ai0 + aj0 + ae0
            s1 = ai1 + aj1 + ae1
            s0 = jnp.where(s0 >= 0.0, s0, s0 * 0.2)
            s1 = jnp.where(s1 >= 0.0, s1, s1 * 0.2)
            m = jnp.maximum(s0, s1)
            e0 = jnp.exp(s0 - m)
            e1 = jnp.exp(s1 - m)
            inv = 1.0 / (e0 + e1)
            a0v = e0 * inv
            a1v = e1 * inv
            for k in range(16):
                r = j * 16 + k
                a0s = a0v[k]
                a1s = a1v[k]
                for q in range(4):
                    rows_v[r, pl.ds(q * 16, 16)] = (
                        rows_v[r, pl.ds(q * 16, 16)] * a0s)
                for q in range(4, 8):
                    rows_v[r, pl.ds(q * 16, 16)] = (
                        rows_v[r, pl.ds(q * 16, 16)] * a1s)
            return carry

        lax.fori_loop(0, _G // 16, jbody, 0)
        pltpu.sync_copy(rows_v, agg_sh.at[dst_v], add=True)

    fetch_idx(0, sets[0])
    issue(sets[0])

    def pair(p, carry):
        for ph in range(2):
            g = 2 * p + ph
            cur = sets[ph]
            nxt = sets[1 - ph]
            wait(cur)
            fetch_idx(g + 1, nxt)
            issue(nxt)
            compute(cur)
        return carry

    lax.fori_loop(0, _GPT // 2 - 1, pair, 0)
    wait(sets[0])
    fetch_idx(_GPT - 1, sets[1])
    issue(sets[1])
    compute(sets[0])
    wait(sets[1])
    compute(sets[1])
    plsc.subcore_barrier()
    start = 8 * (sid * 78 + jnp.minimum(sid, 2))

    @pl.when(sid < 2)
    def _read_wide():
        pltpu.sync_copy(agg_sh.at[pl.ds(start, 632)],
                        out_h.at[pl.ds(cid * _N + start, 632)])

    @pl.when(sid >= 2)
    def _read_narrow():
        pltpu.sync_copy(agg_sh.at[pl.ds(start, 624)],
                        out_h.at[pl.ds(cid * _N + start, 624)])


@functools.cache
def _get_sc_edge():
    return pl.kernel(
        _sc_edge_body,
        out_type=jax.ShapeDtypeStruct((2 * _N, _HC), jnp.float32),
        mesh=plsc.VectorSubcoreMesh(core_axis_name="c", subcore_axis_name="s",
                                    num_cores=_NC, num_subcores=_NS),
        compiler_params=pltpu.CompilerParams(needs_layout_passes=False),
        scratch_types=(
            [pltpu.VMEM((_G,), jnp.int32),
             pltpu.VMEM((_G,), jnp.int32),
             pltpu.VMEM((2 * _G,), jnp.float32),
             pltpu.VMEM((_G, _HC), jnp.float32),
             ] * 2 +
            [pltpu.VMEM((2 * _NP,), jnp.int32),
             pltpu.VMEM_SHARED((_NP, _HC), jnp.float32),
             pltpu.SemaphoreType.DMA,
             pltpu.SemaphoreType.DMA,
             ]),
    )


_BN = 2000
_BE = 4000


def _node_body(x_ref, wt_ref, am_ref, xl_ref, a_ref):
    xl = jnp.dot(x_ref[...], wt_ref[...], preferred_element_type=jnp.float32)
    xl_ref[...] = xl
    a_ref[...] = jnp.dot(xl, am_ref[...], preferred_element_type=jnp.float32)


def _tc_node(x, wt, am):
    return pl.pallas_call(
        _node_body,
        grid=(_N // _BN,),
        in_specs=[
            pl.BlockSpec((_BN, _HC), lambda i: (i, 0)),
            pl.BlockSpec((_HC, _HC), lambda i: (0, 0)),
            pl.BlockSpec((_HC, 4), lambda i: (0, 0)),
        ],
        out_specs=[
            pl.BlockSpec((_BN, _HC), lambda i: (i, 0)),
            pl.BlockSpec((_BN, 4), lambda i: (i, 0)),
        ],
        out_shape=[
            jax.ShapeDtypeStruct((_N, _HC), jnp.float32),
            jax.ShapeDtypeStruct((_N, 4), jnp.float32),
        ],
    )(x, wt, am)


def _combine_body(p_ref, b_ref, wt_ref, am_ref, xl_ref, a_ref):
    h = jnp.maximum(p_ref[0] + p_ref[1] + b_ref[...], 0.0)
    xl = jnp.dot(h, wt_ref[...], preferred_element_type=jnp.float32)
    xl_ref[...] = xl
    a_ref[...] = jnp.dot(xl, am_ref[...], preferred_element_type=jnp.float32)


def _tc_combine(parts, brow, wt, am):
    return pl.pallas_call(
        _combine_body,
        grid=(_N // _BN,),
        in_specs=[
            pl.BlockSpec((2, _BN, _HC), lambda i: (0, i, 0)),
            pl.BlockSpec((1, _HC), lambda i: (0, 0)),
            pl.BlockSpec((_HC, _HC), lambda i: (0, 0)),
            pl.BlockSpec((_HC, 4), lambda i: (0, 0)),
        ],
        out_specs=[
            pl.BlockSpec((_BN, _HC), lambda i: (i, 0)),
            pl.BlockSpec((_BN, 4), lambda i: (i, 0)),
        ],
        out_shape=[
            jax.ShapeDtypeStruct((_N, _HC), jnp.float32),
            jax.ShapeDtypeStruct((_N, 4), jnp.float32),
        ],
    )(parts, brow, wt, am)


def _ea_body(ea_ref, b_ref, o_ref):
    o_ref[...] = jnp.dot(ea_ref[...], b_ref[...],
                         preferred_element_type=jnp.float32)


def _tc_ea(ea_view, bcat):
    e8 = _E // 8
    return pl.pallas_call(
        _ea_body,
        grid=(e8 // _BE,),
        in_specs=[
            pl.BlockSpec((_BE, _HC), lambda i: (i, 0)),
            pl.BlockSpec((_HC, 32), lambda i: (0, 0)),
        ],
        out_specs=pl.BlockSpec((_BE, 32), lambda i: (i, 0)),
        out_shape=jax.ShapeDtypeStruct((e8, 32), jnp.float32),
    )(ea_view, bcat)


def _final_body(p_ref, b_ref, bt_ref, wo_ref, bo_ref, o_ref, sum_acc, cnt_acc):
    i = pl.program_id(0)

    @pl.when(i == 0)
    def _init():
        sum_acc[...] = jnp.zeros_like(sum_acc)
        cnt_acc[...] = jnp.zeros_like(cnt_acc)

    h = jnp.maximum(p_ref[0] + p_ref[1] + b_ref[...], 0.0)
    oh = (bt_ref[...] == lax.broadcasted_iota(jnp.int32, (1, _NG), 1)
          ).astype(jnp.float32)
    dnum = (((0,), (0,)), ((), ()))
    sum_acc[...] += lax.dot_general(oh, h, dnum,
                                    preferred_element_type=jnp.float32)
    cnt_acc[...] += lax.dot_general(oh, jnp.ones((_BN, _HC), jnp.float32),
                                    dnum, preferred_element_type=jnp.float32)

    @pl.when(i == _N // _BN - 1)
    def _fin():
        pooled = sum_acc[...] / jnp.maximum(cnt_acc[...], 1.0)
        logits = jnp.dot(pooled, wo_ref[...],
                         preferred_element_type=jnp.float32) + bo_ref[...]
        m = jnp.max(logits, axis=1, keepdims=True)
        sh = logits - m
        o_ref[...] = sh - jnp.log(jnp.sum(jnp.exp(sh), axis=1, keepdims=True))


def _tc_final(parts, brow, batch2d, wot, borow):
    return pl.pallas_call(
        _final_body,
        grid=(_N // _BN,),
        in_specs=[
            pl.BlockSpec((2, _BN, _HC), lambda i: (0, i, 0)),
            pl.BlockSpec((1, _HC), lambda i: (0, 0)),
            pl.BlockSpec((_BN, 1), lambda i: (i, 0)),
            pl.BlockSpec((_HC, _NCLS), lambda i: (0, 0)),
            pl.BlockSpec((1, _NCLS), lambda i: (0, 0)),
        ],
        out_specs=pl.BlockSpec((_NG, _NCLS), lambda i: (0, 0)),
        out_shape=jax.ShapeDtypeStruct((_NG, _NCLS), jnp.float32),
        scratch_shapes=[
            pltpu.VMEM((_NG, _HC), jnp.float32),
            pltpu.VMEM((_NG, _HC), jnp.float32),
        ],
    )(parts, brow, batch2d, wot, borow)


def _build_A(att):
    a = jnp.zeros((_HC, 4), jnp.float32)
    a = a.at[0:_C, 0].set(att[0, 0, 0:_C])
    a = a.at[_C:_HC, 1].set(att[0, 1, 0:_C])
    a = a.at[0:_C, 2].set(att[0, 0, _C:2 * _C])
    a = a.at[_C:_HC, 3].set(att[0, 1, _C:2 * _C])
    return a


def _build_B(we, att):
    ve = jnp.stack(
        [we[h * _C:(h + 1) * _C, :].T @ att[0, h, 2 * _C:] for h in range(2)],
        axis=1)
    return jnp.kron(jnp.eye(8, dtype=jnp.float32), ve)


def _pack_tbl(a):
    t = lax.bitcast_convert_type(a.astype(jnp.bfloat16).reshape(-1, 2),
                                 jnp.int32)
    return jnp.concatenate([t, jnp.zeros((2 * (_NP - _N),), jnp.int32)])


def kernel(x, edge_index, edge_attr, batch, W0, We0, att0, b0,
           W1, We1, att1, b1, Wout, bout):
    pad_e = _EPAD - _E
    src = jnp.concatenate([edge_index[0].astype(jnp.int32),
                           jnp.full((pad_e,), _N, jnp.int32)])
    dst = jnp.concatenate([edge_index[1].astype(jnp.int32),
                           jnp.full((pad_e,), _N, jnp.int32)])
    batch2d = batch.astype(jnp.int32).reshape(_N, 1)

    A0 = _build_A(att0)
    A1 = _build_A(att1)
    bcat = jnp.concatenate([_build_B(We0, att0), _build_B(We1, att1)], axis=1)
    ea_view = edge_attr.reshape(_E // 8, _HC)
    zrows = jnp.zeros((_ZROWS, _HC), jnp.float32)
    ae_pad = jnp.zeros((2 * pad_e,), jnp.float32)
    xl_pad = jnp.zeros((_NP - _N, _HC), jnp.float32)

    ae_all = _tc_ea(ea_view, bcat)
    ae0 = jnp.concatenate([ae_all[:, :16].reshape(-1), ae_pad])
    ae1 = jnp.concatenate([ae_all[:, 16:].reshape(-1), ae_pad])

    sc_edge = _get_sc_edge()
    xl0, a0 = _tc_node(x, W0.T, A0)
    parts0 = sc_edge(src, dst, _pack_tbl(a0), ae0,
                     jnp.concatenate([xl0, xl_pad]),
                     zrows).reshape(2, _N, _HC)
    xl1, a1 = _tc_combine(parts0, b0.reshape(1, _HC), W1.T, A1)
    parts1 = sc_edge(src, dst, _pack_tbl(a1), ae1,
                     jnp.concatenate([xl1, xl_pad]),
                     zrows).reshape(2, _N, _HC)
    return _tc_final(parts1, b1.reshape(1, _HC), batch2d,
                     Wout.T, bout.reshape(1, _NCLS))


# speedup vs baseline: 68.6025x; 1.0391x over previous
import functools

import jax
import jax.numpy as jnp
from jax import lax
from jax.experimental import pallas as pl
from jax.experimental.pallas import tpu as pltpu
from jax.experimental.pallas import tpu_sc as plsc

_N = 10000
_E = 320000
_HC = 128
_C = 64
_DE = 16
_NG = 16
_NCLS = 4

_NC = 2
_NS = 16
_NW = _NC * _NS
_G = 112
_GPT = 90
_EPT = _G * _GPT
_EPAD = _NW * _EPT
_NP = _N + 8
_ZROWS = 640


def _sc_edge_body(src_h, dst_h, tbl_h, ae_h, xl_h, z_h, out_h,
                  src_v0, dst_v0, ae_v0, rows_v0,
                  src_v1, dst_v1, ae_v1, rows_v1,
                  tbl_v, agg_sh, sem0, sem1, ssem0, ssem1):
    cid = lax.axis_index("c")
    sid = lax.axis_index("s")
    wid = sid * _NC + cid

    pltpu.sync_copy(z_h, agg_sh.at[pl.ds(sid * 624, _ZROWS)])
    pltpu.sync_copy(tbl_h, tbl_v)
    plsc.subcore_barrier()

    ebase = wid * _EPT
    sets = ((src_v0, dst_v0, ae_v0, rows_v0, sem0, ssem0),
            (src_v1, dst_v1, ae_v1, rows_v1, sem1, ssem1))

    def fetch_idx(g, s):
        base = pl.multiple_of(ebase + g * _G, 16)
        pltpu.sync_copy(src_h.at[pl.ds(base, _G)], s[0])
        pltpu.sync_copy(dst_h.at[pl.ds(base, _G)], s[1])
        pltpu.sync_copy(ae_h.at[pl.ds(base * 2, 2 * _G)], s[2])

    def issue(s):
        pltpu.async_copy(xl_h.at[s[0]], s[3], s[4])

    def wait(s):
        pltpu.make_async_copy(xl_h.at[s[0]], s[3], s[4]).wait()

    def issue_scatter(s):
        pltpu.async_copy(s[3], agg_sh.at[s[1]], s[5], add=True)

    def wait_scatter(s):
        pltpu.make_async_copy(s[3], agg_sh.at[s[1]], s[5]).wait()

    def compute(s):
        src_v, dst_v, ae_v, rows_v = s[:4]
        lane = lax.iota(jnp.int32, 16)

        def jbody(j, carry):
            sl = pl.ds(j * 16, 16)
            s16 = src_v[sl]
            d16 = dst_v[sl]
            wi = plsc.load_gather(tbl_v, [d16 * 2])
            wj = plsc.load_gather(tbl_v, [s16 * 2 + 1])
            ai0, ai1 = plsc.unpack(plsc.bitcast(wi, jnp.bfloat16),
                                   format=plsc.PackFormat.INTERLEAVED)
            aj0, aj1 = plsc.unpack(plsc.bitcast(wj, jnp.bfloat16),
                                   format=plsc.PackFormat.INTERLEAVED)
            ei = j * 32 + lane * 2
            ae0 = plsc.load_gather(ae_v, [ei])
            ae1 = plsc.load_gather(ae_v, [ei + 1])
            s0 = ai0 + aj0 + ae0
            s1 = ai1 + aj1 + ae1
            s0 = jnp.where(s0 >= 0.0, s0, s0 * 0.2)
            s1 = jnp.where(s1 >= 0.0, s1, s1 * 0.2)
            m = jnp.maximum(s0, s1)
            e0 = jnp.exp(s0 - m)
            e1 = jnp.exp(s1 - m)
            inv = 1.0 / (e0 + e1)
            a0v = e0 * inv
            a1v = e1 * inv
            for k in range(16):
                r = j * 16 + k
                a0s = a0v[k]
                a1s = a1v[k]
                for q in range(4):
                    rows_v[r, pl.ds(q * 16, 16)] = (
                        rows_v[r, pl.ds(q * 16, 16)] * a0s)
                for q in range(4, 8):
                    rows_v[r, pl.ds(q * 16, 16)] = (
                        rows_v[r, pl.ds(q * 16, 16)] * a1s)
            return carry

        lax.fori_loop(0, _G // 16, jbody, 0)

    fetch_idx(0, sets[0])
    issue(sets[0])
    wait(sets[0])
    fetch_idx(1, sets[1])
    issue(sets[1])
    compute(sets[0])
    issue_scatter(sets[0])

    def pair(p, carry):
        for ph in range(2):
            g = 1 + 2 * p + ph
            cur = sets[(1 + ph) % 2]
            nxt = sets[ph]
            wait(cur)
            wait_scatter(nxt)
            fetch_idx(g + 1, nxt)
            issue(nxt)
            compute(cur)
            issue_scatter(cur)
        return carry

    lax.fori_loop(0, _GPT // 2 - 1, pair, 0)
    wait(sets[1])
    compute(sets[1])
    issue_scatter(sets[1])
    wait_scatter(sets[0])
    wait_scatter(sets[1])
    plsc.subcore_barrier()
    start = 8 * (sid * 78 + jnp.minimum(sid, 2))

    @pl.when(sid < 2)
    def _read_wide():
        pltpu.sync_copy(agg_sh.at[pl.ds(start, 632)],
                        out_h.at[pl.ds(cid * _N + start, 632)])

    @pl.when(sid >= 2)
    def _read_narrow():
        pltpu.sync_copy(agg_sh.at[pl.ds(start, 624)],
                        out_h.at[pl.ds(cid * _N + start, 624)])


@functools.cache
def _get_sc_edge():
    return pl.kernel(
        _sc_edge_body,
        out_type=jax.ShapeDtypeStruct((2 * _N, _HC), jnp.float32),
        mesh=plsc.VectorSubcoreMesh(core_axis_name="c", subcore_axis_name="s",
                                    num_cores=_NC, num_subcores=_NS),
        compiler_params=pltpu.CompilerParams(needs_layout_passes=False),
        scratch_types=(
            [pltpu.VMEM((_G,), jnp.int32),
             pltpu.VMEM((_G,), jnp.int32),
             pltpu.VMEM((2 * _G,), jnp.float32),
             pltpu.VMEM((_G, _HC), jnp.float32),
             ] * 2 +
            [pltpu.VMEM((2 * _NP,), jnp.int32),
             pltpu.VMEM_SHARED((_NP, _HC), jnp.float32),
             pltpu.SemaphoreType.DMA,
             pltpu.SemaphoreType.DMA,
             pltpu.SemaphoreType.DMA,
             pltpu.SemaphoreType.DMA,
             ]),
    )


_BN = 2000
_BE = 4000


def _node_body(x_ref, wt_ref, am_ref, xl_ref, a_ref):
    xl = jnp.dot(x_ref[...], wt_ref[...], preferred_element_type=jnp.float32)
    xl_ref[...] = xl
    a_ref[...] = jnp.dot(xl, am_ref[...], preferred_element_type=jnp.float32)


def _tc_node(x, wt, am):
    return pl.pallas_call(
        _node_body,
        grid=(_N // _BN,),
        in_specs=[
            pl.BlockSpec((_BN, _HC), lambda i: (i, 0)),
            pl.BlockSpec((_HC, _HC), lambda i: (0, 0)),
            pl.BlockSpec((_HC, 4), lambda i: (0, 0)),
        ],
        out_specs=[
            pl.BlockSpec((_BN, _HC), lambda i: (i, 0)),
            pl.BlockSpec((_BN, 4), lambda i: (i, 0)),
        ],
        out_shape=[
            jax.ShapeDtypeStruct((_NP, _HC), jnp.float32),
            jax.ShapeDtypeStruct((_N, 4), jnp.float32),
        ],
    )(x, wt, am)


def _combine_body(p_ref, b_ref, wt_ref, am_ref, xl_ref, a_ref):
    h = jnp.maximum(p_ref[0] + p_ref[1] + b_ref[...], 0.0)
    xl = jnp.dot(h, wt_ref[...], preferred_element_type=jnp.float32)
    xl_ref[...] = xl
    a_ref[...] = jnp.dot(xl, am_ref[...], preferred_element_type=jnp.float32)


def _tc_combine(parts, brow, wt, am):
    return pl.pallas_call(
        _combine_body,
        grid=(_N // _BN,),
        in_specs=[
            pl.BlockSpec((2, _BN, _HC), lambda i: (0, i, 0)),
            pl.BlockSpec((1, _HC), lambda i: (0, 0)),
            pl.BlockSpec((_HC, _HC), lambda i: (0, 0)),
            pl.BlockSpec((_HC, 4), lambda i: (0, 0)),
        ],
        out_specs=[
            pl.BlockSpec((_BN, _HC), lambda i: (i, 0)),
            pl.BlockSpec((_BN, 4), lambda i: (i, 0)),
        ],
        out_shape=[
            jax.ShapeDtypeStruct((_NP, _HC), jnp.float32),
            jax.ShapeDtypeStruct((_N, 4), jnp.float32),
        ],
    )(parts, brow, wt, am)


def _ea_body(ea_ref, b_ref, o_ref):
    o_ref[...] = jnp.dot(ea_ref[...], b_ref[...],
                         preferred_element_type=jnp.float32)


def _tc_ea(ea_view, bcat):
    e8 = _E // 8
    return pl.pallas_call(
        _ea_body,
        grid=(e8 // _BE,),
        in_specs=[
            pl.BlockSpec((_BE, _HC), lambda i: (i, 0)),
            pl.BlockSpec((_HC, 32), lambda i: (0, 0)),
        ],
        out_specs=pl.BlockSpec((_BE, 32), lambda i: (i, 0)),
        out_shape=jax.ShapeDtypeStruct((e8, 32), jnp.float32),
    )(ea_view, bcat)


def _final_body(p_ref, b_ref, bt_ref, wo_ref, bo_ref, o_ref, sum_acc, cnt_acc):
    i = pl.program_id(0)

    @pl.when(i == 0)
    def _init():
        sum_acc[...] = jnp.zeros_like(sum_acc)
        cnt_acc[...] = jnp.zeros_like(cnt_acc)

    h = jnp.maximum(p_ref[0] + p_ref[1] + b_ref[...], 0.0)
    oh = (bt_ref[...] == lax.broadcasted_iota(jnp.int32, (1, _NG), 1)
          ).astype(jnp.float32)
    dnum = (((0,), (0,)), ((), ()))
    sum_acc[...] += lax.dot_general(oh, h, dnum,
                                    preferred_element_type=jnp.float32)
    cnt_acc[...] += lax.dot_general(oh, jnp.ones((_BN, _HC), jnp.float32),
                                    dnum, preferred_element_type=jnp.float32)

    @pl.when(i == _N // _BN - 1)
    def _fin():
        pooled = sum_acc[...] / jnp.maximum(cnt_acc[...], 1.0)
        logits = jnp.dot(pooled, wo_ref[...],
                         preferred_element_type=jnp.float32) + bo_ref[...]
        m = jnp.max(logits, axis=1, keepdims=True)
        sh = logits - m
        o_ref[...] = sh - jnp.log(jnp.sum(jnp.exp(sh), axis=1, keepdims=True))


def _tc_final(parts, brow, batch2d, wot, borow):
    return pl.pallas_call(
        _final_body,
        grid=(_N // _BN,),
        in_specs=[
            pl.BlockSpec((2, _BN, _HC), lambda i: (0, i, 0)),
            pl.BlockSpec((1, _HC), lambda i: (0, 0)),
            pl.BlockSpec((_BN, 1), lambda i: (i, 0)),
            pl.BlockSpec((_HC, _NCLS), lambda i: (0, 0)),
            pl.BlockSpec((1, _NCLS), lambda i: (0, 0)),
        ],
        out_specs=pl.BlockSpec((_NG, _NCLS), lambda i: (0, 0)),
        out_shape=jax.ShapeDtypeStruct((_NG, _NCLS), jnp.float32),
        scratch_shapes=[
            pltpu.VMEM((_NG, _HC), jnp.float32),
            pltpu.VMEM((_NG, _HC), jnp.float32),
        ],
    )(parts, brow, batch2d, wot, borow)


def _build_A(att):
    a = jnp.zeros((_HC, 4), jnp.float32)
    a = a.at[0:_C, 0].set(att[0, 0, 0:_C])
    a = a.at[_C:_HC, 1].set(att[0, 1, 0:_C])
    a = a.at[0:_C, 2].set(att[0, 0, _C:2 * _C])
    a = a.at[_C:_HC, 3].set(att[0, 1, _C:2 * _C])
    return a


def _build_B(we, att):
    ve = jnp.stack(
        [we[h * _C:(h + 1) * _C, :].T @ att[0, h, 2 * _C:] for h in range(2)],
        axis=1)
    return jnp.kron(jnp.eye(8, dtype=jnp.float32), ve)


def _pack_tbl(a):
    t = lax.bitcast_convert_type(a.astype(jnp.bfloat16).reshape(-1, 2),
                                 jnp.int32)
    return jnp.concatenate([t, jnp.zeros((2 * (_NP - _N),), jnp.int32)])


def kernel(x, edge_index, edge_attr, batch, W0, We0, att0, b0,
           W1, We1, att1, b1, Wout, bout):
    pad_e = _EPAD - _E
    src = jnp.concatenate([edge_index[0].astype(jnp.int32),
                           jnp.full((pad_e,), _N, jnp.int32)])
    dst = jnp.concatenate([edge_index[1].astype(jnp.int32),
                           jnp.full((pad_e,), _N, jnp.int32)])
    batch2d = batch.astype(jnp.int32).reshape(_N, 1)

    A0 = _build_A(att0)
    A1 = _build_A(att1)
    bcat = jnp.concatenate([_build_B(We0, att0), _build_B(We1, att1)], axis=1)
    ea_view = edge_attr.reshape(_E // 8, _HC)
    zrows = jnp.zeros((_ZROWS, _HC), jnp.float32)
    ae_pad = jnp.zeros((2 * pad_e,), jnp.float32)

    ae_all = _tc_ea(ea_view, bcat)
    ae0 = jnp.concatenate([ae_all[:, :16].reshape(-1), ae_pad])
    ae1 = jnp.concatenate([ae_all[:, 16:].reshape(-1), ae_pad])

    sc_edge = _get_sc_edge()
    xl0, a0 = _tc_node(x, W0.T, A0)
    parts0 = sc_edge(src, dst, _pack_tbl(a0), ae0, xl0,
                     zrows).reshape(2, _N, _HC)
    xl1, a1 = _tc_combine(parts0, b0.reshape(1, _HC), W1.T, A1)
    parts1 = sc_edge(src, dst, _pack_tbl(a1), ae1, xl1,
                     zrows).reshape(2, _N, _HC)
    return _tc_final(parts1, b1.reshape(1, _HC), batch2d,
                     Wout.T, bout.reshape(1, _NCLS))


# speedup vs baseline: 88.6906x; 1.2928x over previous
import functools

import jax
import jax.numpy as jnp
from jax import lax
from jax.experimental import pallas as pl
from jax.experimental.pallas import tpu as pltpu
from jax.experimental.pallas import tpu_sc as plsc

_N = 10000
_E = 320000
_HC = 128
_C = 64
_DE = 16
_NG = 16
_NCLS = 4

_NC = 2
_NS = 16
_NW = _NC * _NS
_G = 112
_GPT = 90
_EPT = _G * _GPT
_EPAD = _NW * _EPT
_NP = _N + 8
_ZROWS = 640


def _sc_edge_body(src_h, dst_h, tbl_h, ae_h, xl_h, z_h, out_h,
                  rows_v0, rows_v1,
                  srcq0, dstq0, aeq0, srcq1, dstq1, aeq1,
                  srcq2, dstq2, aeq2, srcq3, dstq3, aeq3,
                  tbl_v, agg_sh,
                  gsem0, gsem1, ssem0, ssem1,
                  isem0, isem1, isem2, isem3):
    cid = lax.axis_index("c")
    sid = lax.axis_index("s")
    wid = sid * _NC + cid

    pltpu.sync_copy(z_h, agg_sh.at[pl.ds(sid * 624, _ZROWS)])
    pltpu.sync_copy(tbl_h, tbl_v)
    plsc.subcore_barrier()

    ebase = wid * _EPT
    rows = ((rows_v0, gsem0, ssem0), (rows_v1, gsem1, ssem1))
    slots = ((srcq0, dstq0, aeq0, isem0), (srcq1, dstq1, aeq1, isem1),
             (srcq2, dstq2, aeq2, isem2), (srcq3, dstq3, aeq3, isem3))

    def idx_copies(g, q):
        base = pl.multiple_of(ebase + g * _G, 16)
        return (pltpu.make_async_copy(src_h.at[pl.ds(base, _G)], q[0], q[3]),
                pltpu.make_async_copy(dst_h.at[pl.ds(base, _G)], q[1], q[3]),
                pltpu.make_async_copy(ae_h.at[pl.ds(base * 2, 2 * _G)],
                                      q[2], q[3]))

    def fetch_idx(g, q):
        for c in idx_copies(g, q):
            c.start()

    def wait_idx(q):
        for c in idx_copies(0, q):
            c.wait()

    def issue_gather(r, q):
        pltpu.async_copy(xl_h.at[q[0]], r[0], r[1])

    def wait_gather(r, q):
        pltpu.make_async_copy(xl_h.at[q[0]], r[0], r[1]).wait()

    def issue_scatter(r, q):
        pltpu.async_copy(r[0], agg_sh.at[q[1]], r[2], add=True)

    def wait_scatter(r, q):
        pltpu.make_async_copy(r[0], agg_sh.at[q[1]], r[2]).wait()

    def compute(r, q):
        src_v, dst_v, ae_v = q[0], q[1], q[2]
        rows_v = r[0]
        lane = lax.iota(jnp.int32, 16)

        def jbody(j, carry):
            sl = pl.ds(j * 16, 16)
            s16 = src_v[sl]
            d16 = dst_v[sl]
            wi = plsc.load_gather(tbl_v, [d16 * 2])
            wj = plsc.load_gather(tbl_v, [s16 * 2 + 1])
            ai0, ai1 = plsc.unpack(plsc.bitcast(wi, jnp.bfloat16),
                                   format=plsc.PackFormat.INTERLEAVED)
            aj0, aj1 = plsc.unpack(plsc.bitcast(wj, jnp.bfloat16),
                                   format=plsc.PackFormat.INTERLEAVED)
            ei = j * 32 + lane * 2
            ae0 = plsc.load_gather(ae_v, [ei])
            ae1 = plsc.load_gather(ae_v, [ei + 1])
            s0 = ai0 + aj0 + ae0
            s1 = ai1 + aj1 + ae1
            s0 = jnp.where(s0 >= 0.0, s0, s0 * 0.2)
            s1 = jnp.where(s1 >= 0.0, s1, s1 * 0.2)
            m = jnp.maximum(s0, s1)
            e0 = jnp.exp(s0 - m)
            e1 = jnp.exp(s1 - m)
            inv = 1.0 / (e0 + e1)
            a0v = e0 * inv
            a1v = e1 * inv
            for k in range(16):
                r = j * 16 + k
                a0s = a0v[k]
                a1s = a1v[k]
                for q in range(4):
                    rows_v[r, pl.ds(q * 16, 16)] = (
                        rows_v[r, pl.ds(q * 16, 16)] * a0s)
                for q in range(4, 8):
                    rows_v[r, pl.ds(q * 16, 16)] = (
                        rows_v[r, pl.ds(q * 16, 16)] * a1s)
            return carry

        lax.fori_loop(0, _G // 16, jbody, 0)

    def phase(g, r4, has_scatter_wait=True, do_fetch=True):
        s = r4 % 2
        wait_idx(slots[(r4 + 1) % 4])
        if has_scatter_wait:
            wait_scatter(rows[1 - s], slots[(r4 - 1) % 4])
        issue_gather(rows[1 - s], slots[(r4 + 1) % 4])
        if do_fetch:
            fetch_idx(g + 2, slots[(r4 + 2) % 4])
        wait_gather(rows[s], slots[r4])
        compute(rows[s], slots[r4])
        issue_scatter(rows[s], slots[r4])

    fetch_idx(0, slots[0])
    fetch_idx(1, slots[1])
    wait_idx(slots[0])
    issue_gather(rows[0], slots[0])
    phase(0, 0, has_scatter_wait=False)
    phase(1, 1)

    def quad(p, carry):
        for i in range(4):
            phase(2 + 4 * p + i, (2 + i) % 4)
        return carry

    lax.fori_loop(0, (_GPT - 6) // 4, quad, 0)
    phase(_GPT - 4, (_GPT - 4) % 4)
    phase(_GPT - 3, (_GPT - 3) % 4)
    phase(_GPT - 2, (_GPT - 2) % 4, do_fetch=False)
    wait_gather(rows[(_GPT - 1) % 2], slots[(_GPT - 1) % 4])
    compute(rows[(_GPT - 1) % 2], slots[(_GPT - 1) % 4])
    issue_scatter(rows[(_GPT - 1) % 2], slots[(_GPT - 1) % 4])
    wait_scatter(rows[0], slots[(_GPT - 2) % 4])
    wait_scatter(rows[1], slots[(_GPT - 1) % 4])
    plsc.subcore_barrier()
    start = 8 * (sid * 78 + jnp.minimum(sid, 2))

    @pl.when(sid < 2)
    def _read_wide():
        pltpu.sync_copy(agg_sh.at[pl.ds(start, 632)],
                        out_h.at[pl.ds(cid * _N + start, 632)])

    @pl.when(sid >= 2)
    def _read_narrow():
        pltpu.sync_copy(agg_sh.at[pl.ds(start, 624)],
                        out_h.at[pl.ds(cid * _N + start, 624)])


@functools.cache
def _get_sc_edge():
    return pl.kernel(
        _sc_edge_body,
        out_type=jax.ShapeDtypeStruct((2 * _N, _HC), jnp.float32),
        mesh=plsc.VectorSubcoreMesh(core_axis_name="c", subcore_axis_name="s",
                                    num_cores=_NC, num_subcores=_NS),
        compiler_params=pltpu.CompilerParams(needs_layout_passes=False),
        scratch_types=(
            [pltpu.VMEM((_G, _HC), jnp.float32),
             ] * 2 +
            [pltpu.VMEM((_G,), jnp.int32),
             pltpu.VMEM((_G,), jnp.int32),
             pltpu.VMEM((2 * _G,), jnp.float32),
             ] * 4 +
            [pltpu.VMEM((2 * _NP,), jnp.int32),
             pltpu.VMEM_SHARED((_NP, _HC), jnp.float32),
             ] +
            [pltpu.SemaphoreType.DMA] * 8),
    )


_BN = 2000
_BE = 4000


def _node_body(x_ref, wt_ref, am_ref, xl_ref, a_ref):
    xl = jnp.dot(x_ref[...], wt_ref[...], preferred_element_type=jnp.float32)
    xl_ref[...] = xl
    a_ref[...] = jnp.dot(xl, am_ref[...], preferred_element_type=jnp.float32)


def _tc_node(x, wt, am):
    return pl.pallas_call(
        _node_body,
        grid=(_N // _BN,),
        in_specs=[
            pl.BlockSpec((_BN, _HC), lambda i: (i, 0)),
            pl.BlockSpec((_HC, _HC), lambda i: (0, 0)),
            pl.BlockSpec((_HC, 4), lambda i: (0, 0)),
        ],
        out_specs=[
            pl.BlockSpec((_BN, _HC), lambda i: (i, 0)),
            pl.BlockSpec((_BN, 4), lambda i: (i, 0)),
        ],
        out_shape=[
            jax.ShapeDtypeStruct((_NP, _HC), jnp.float32),
            jax.ShapeDtypeStruct((_N, 4), jnp.float32),
        ],
    )(x, wt, am)


def _combine_body(p_ref, b_ref, wt_ref, am_ref, xl_ref, a_ref):
    h = jnp.maximum(p_ref[0] + p_ref[1] + b_ref[...], 0.0)
    xl = jnp.dot(h, wt_ref[...], preferred_element_type=jnp.float32)
    xl_ref[...] = xl
    a_ref[...] = jnp.dot(xl, am_ref[...], preferred_element_type=jnp.float32)


def _tc_combine(parts, brow, wt, am):
    return pl.pallas_call(
        _combine_body,
        grid=(_N // _BN,),
        in_specs=[
            pl.BlockSpec((2, _BN, _HC), lambda i: (0, i, 0)),
            pl.BlockSpec((1, _HC), lambda i: (0, 0)),
            pl.BlockSpec((_HC, _HC), lambda i: (0, 0)),
            pl.BlockSpec((_HC, 4), lambda i: (0, 0)),
        ],
        out_specs=[
            pl.BlockSpec((_BN, _HC), lambda i: (i, 0)),
            pl.BlockSpec((_BN, 4), lambda i: (i, 0)),
        ],
        out_shape=[
            jax.ShapeDtypeStruct((_NP, _HC), jnp.float32),
            jax.ShapeDtypeStruct((_N, 4), jnp.float32),
        ],
    )(parts, brow, wt, am)


def _ea_body(ea_ref, b_ref, o_ref):
    o_ref[...] = jnp.dot(ea_ref[...], b_ref[...],
                         preferred_element_type=jnp.float32)


def _tc_ea(ea_view, bcat):
    e8 = _E // 8
    return pl.pallas_call(
        _ea_body,
        grid=(e8 // _BE,),
        in_specs=[
            pl.BlockSpec((_BE, _HC), lambda i: (i, 0)),
            pl.BlockSpec((_HC, 32), lambda i: (0, 0)),
        ],
        out_specs=pl.BlockSpec((_BE, 32), lambda i: (i, 0)),
        out_shape=jax.ShapeDtypeStruct((e8, 32), jnp.float32),
    )(ea_view, bcat)


def _final_body(p_ref, b_ref, bt_ref, wo_ref, bo_ref, o_ref, sum_acc, cnt_acc):
    i = pl.program_id(0)

    @pl.when(i == 0)
    def _init():
        sum_acc[...] = jnp.zeros_like(sum_acc)
        cnt_acc[...] = jnp.zeros_like(cnt_acc)

    h = jnp.maximum(p_ref[0] + p_ref[1] + b_ref[...], 0.0)
    oh = (bt_ref[...] == lax.broadcasted_iota(jnp.int32, (1, _NG), 1)
          ).astype(jnp.float32)
    dnum = (((0,), (0,)), ((), ()))
    sum_acc[...] += lax.dot_general(oh, h, dnum,
                                    preferred_element_type=jnp.float32)
    cnt_acc[...] += lax.dot_general(oh, jnp.ones((_BN, _HC), jnp.float32),
                                    dnum, preferred_element_type=jnp.float32)

    @pl.when(i == _N // _BN - 1)
    def _fin():
        pooled = sum_acc[...] / jnp.maximum(cnt_acc[...], 1.0)
        logits = jnp.dot(pooled, wo_ref[...],
                         preferred_element_type=jnp.float32) + bo_ref[...]
        m = jnp.max(logits, axis=1, keepdims=True)
        sh = logits - m
        o_ref[...] = sh - jnp.log(jnp.sum(jnp.exp(sh), axis=1, keepdims=True))


def _tc_final(parts, brow, batch2d, wot, borow):
    return pl.pallas_call(
        _final_body,
        grid=(_N // _BN,),
        in_specs=[
            pl.BlockSpec((2, _BN, _HC), lambda i: (0, i, 0)),
            pl.BlockSpec((1, _HC), lambda i: (0, 0)),
            pl.BlockSpec((_BN, 1), lambda i: (i, 0)),
            pl.BlockSpec((_HC, _NCLS), lambda i: (0, 0)),
            pl.BlockSpec((1, _NCLS), lambda i: (0, 0)),
        ],
        out_specs=pl.BlockSpec((_NG, _NCLS), lambda i: (0, 0)),
        out_shape=jax.ShapeDtypeStruct((_NG, _NCLS), jnp.float32),
        scratch_shapes=[
            pltpu.VMEM((_NG, _HC), jnp.float32),
            pltpu.VMEM((_NG, _HC), jnp.float32),
        ],
    )(parts, brow, batch2d, wot, borow)


def _build_A(att):
    a = jnp.zeros((_HC, 4), jnp.float32)
    a = a.at[0:_C, 0].set(att[0, 0, 0:_C])
    a = a.at[_C:_HC, 1].set(att[0, 1, 0:_C])
    a = a.at[0:_C, 2].set(att[0, 0, _C:2 * _C])
    a = a.at[_C:_HC, 3].set(att[0, 1, _C:2 * _C])
    return a


def _build_B(we, att):
    ve = jnp.stack(
        [we[h * _C:(h + 1) * _C, :].T @ att[0, h, 2 * _C:] for h in range(2)],
        axis=1)
    return jnp.kron(jnp.eye(8, dtype=jnp.float32), ve)


def _pack_tbl(a):
    t = lax.bitcast_convert_type(a.astype(jnp.bfloat16).reshape(-1, 2),
                                 jnp.int32)
    return jnp.concatenate([t, jnp.zeros((2 * (_NP - _N),), jnp.int32)])


def kernel(x, edge_index, edge_attr, batch, W0, We0, att0, b0,
           W1, We1, att1, b1, Wout, bout):
    pad_e = _EPAD - _E
    src = jnp.concatenate([edge_index[0].astype(jnp.int32),
                           jnp.full((pad_e,), _N, jnp.int32)])
    dst = jnp.concatenate([edge_index[1].astype(jnp.int32),
                           jnp.full((pad_e,), _N, jnp.int32)])
    batch2d = batch.astype(jnp.int32).reshape(_N, 1)

    A0 = _build_A(att0)
    A1 = _build_A(att1)
    bcat = jnp.concatenate([_build_B(We0, att0), _build_B(We1, att1)], axis=1)
    ea_view = edge_attr.reshape(_E // 8, _HC)
    zrows = jnp.zeros((_ZROWS, _HC), jnp.float32)
    ae_pad = jnp.zeros((2 * pad_e,), jnp.float32)

    ae_all = _tc_ea(ea_view, bcat)
    ae0 = jnp.concatenate([ae_all[:, :16].reshape(-1), ae_pad])
    ae1 = jnp.concatenate([ae_all[:, 16:].reshape(-1), ae_pad])

    sc_edge = _get_sc_edge()
    xl0, a0 = _tc_node(x, W0.T, A0)
    parts0 = sc_edge(src, dst, _pack_tbl(a0), ae0, xl0,
                     zrows).reshape(2, _N, _HC)
    xl1, a1 = _tc_combine(parts0, b0.reshape(1, _HC), W1.T, A1)
    parts1 = sc_edge(src, dst, _pack_tbl(a1), ae1, xl1,
                     zrows).reshape(2, _N, _HC)
    return _tc_final(parts1, b1.reshape(1, _HC), batch2d,
                     Wout.T, bout.reshape(1, _NCLS))


# speedup vs baseline: 91.2222x; 1.0285x over previous
import functools

import jax
import jax.numpy as jnp
from jax import lax
from jax.experimental import pallas as pl
from jax.experimental.pallas import tpu as pltpu
from jax.experimental.pallas import tpu_sc as plsc

_N = 10000
_E = 320000
_HC = 128
_C = 64
_DE = 16
_NG = 16
_NCLS = 4

_NC = 2
_NS = 16
_NW = _NC * _NS
_G = 112
_GPT = 90
_EPT = _G * _GPT
_EPAD = _NW * _EPT
_NP = _N + 8
_ZROWS = 640


def _sc_edge_body(src_h, dst_h, tbl_h, ae_h, xl_h, z_h, out_h,
                  rows_v0, rows_v1,
                  srcq0, dstq0, aeq0, srcq1, dstq1, aeq1,
                  srcq2, dstq2, aeq2, srcq3, dstq3, aeq3,
                  tbl_v, agg_sh,
                  gsem0, gsem1, ssem0, ssem1,
                  isem0, isem1, isem2, isem3):
    cid = lax.axis_index("c")
    sid = lax.axis_index("s")
    wid = sid * _NC + cid

    pltpu.sync_copy(z_h, agg_sh.at[pl.ds(sid * 624, _ZROWS)])
    pltpu.sync_copy(tbl_h, tbl_v)
    plsc.subcore_barrier()

    ebase = wid * _EPT
    rows = ((rows_v0, gsem0, ssem0), (rows_v1, gsem1, ssem1))
    slots = ((srcq0, dstq0, aeq0, isem0), (srcq1, dstq1, aeq1, isem1),
             (srcq2, dstq2, aeq2, isem2), (srcq3, dstq3, aeq3, isem3))

    def idx_copies(g, q):
        base = pl.multiple_of(ebase + g * _G, 16)
        return (pltpu.make_async_copy(src_h.at[pl.ds(base, _G)], q[0], q[3]),
                pltpu.make_async_copy(dst_h.at[pl.ds(base, _G)], q[1], q[3]),
                pltpu.make_async_copy(ae_h.at[pl.ds(base * 2, 2 * _G)],
                                      q[2], q[3]))

    def fetch_idx(g, q):
        for c in idx_copies(g, q):
            c.start()

    def wait_idx(q):
        for c in idx_copies(0, q):
            c.wait()

    def issue_gather(r, q):
        pltpu.async_copy(xl_h.at[q[0]], r[0], r[1])

    def wait_gather(r, q):
        pltpu.make_async_copy(xl_h.at[q[0]], r[0], r[1]).wait()

    def issue_scatter(r, q):
        pltpu.async_copy(r[0], agg_sh.at[q[1]], r[2], add=True)

    def wait_scatter(r, q):
        pltpu.make_async_copy(r[0], agg_sh.at[q[1]], r[2]).wait()

    def compute(r, q):
        src_v, dst_v, ae_v = q[0], q[1], q[2]
        rows_v = r[0]
        lane = lax.iota(jnp.int32, 16)

        def jbody(j, carry):
            sl = pl.ds(j * 16, 16)
            s16 = src_v[sl]
            d16 = dst_v[sl]
            wi = plsc.load_gather(tbl_v, [d16 * 2])
            wj = plsc.load_gather(tbl_v, [s16 * 2 + 1])
            ai0, ai1 = plsc.unpack(plsc.bitcast(wi, jnp.bfloat16),
                                   format=plsc.PackFormat.INTERLEAVED)
            aj0, aj1 = plsc.unpack(plsc.bitcast(wj, jnp.bfloat16),
                                   format=plsc.PackFormat.INTERLEAVED)
            ei = j * 32 + lane * 2
            ae0 = plsc.load_gather(ae_v, [ei])
            ae1 = plsc.load_gather(ae_v, [ei + 1])
            s0 = ai0 + aj0 + ae0
            s1 = ai1 + aj1 + ae1
            s0 = jnp.where(s0 >= 0.0, s0, s0 * 0.2)
            s1 = jnp.where(s1 >= 0.0, s1, s1 * 0.2)
            m = jnp.maximum(s0, s1)
            e0 = jnp.exp(s0 - m)
            e1 = jnp.exp(s1 - m)
            inv = 1.0 / (e0 + e1)
            a0v = e0 * inv
            a1v = e1 * inv
            for k in range(16):
                r = j * 16 + k
                a0s = a0v[k]
                a1s = a1v[k]
                for q in range(4):
                    rows_v[r, pl.ds(q * 16, 16)] = (
                        rows_v[r, pl.ds(q * 16, 16)] * a0s)
                for q in range(4, 8):
                    rows_v[r, pl.ds(q * 16, 16)] = (
                        rows_v[r, pl.ds(q * 16, 16)] * a1s)
            return carry

        lax.fori_loop(0, _G // 16, jbody, 0)

    def phase(g, r4, has_scatter_wait=True, do_fetch=True):
        s = r4 % 2
        wait_idx(slots[(r4 + 1) % 4])
        if has_scatter_wait:
            wait_scatter(rows[1 - s], slots[(r4 - 1) % 4])
        issue_gather(rows[1 - s], slots[(r4 + 1) % 4])
        if do_fetch:
            fetch_idx(g + 2, slots[(r4 + 2) % 4])
        wait_gather(rows[s], slots[r4])
        compute(rows[s], slots[r4])
        issue_scatter(rows[s], slots[r4])

    fetch_idx(0, slots[0])
    fetch_idx(1, slots[1])
    wait_idx(slots[0])
    issue_gather(rows[0], slots[0])
    phase(0, 0, has_scatter_wait=False)
    phase(1, 1)

    def quad(p, carry):
        for i in range(4):
            phase(2 + 4 * p + i, (2 + i) % 4)
        return carry

    lax.fori_loop(0, (_GPT - 6) // 4, quad, 0)
    phase(_GPT - 4, (_GPT - 4) % 4)
    phase(_GPT - 3, (_GPT - 3) % 4)
    phase(_GPT - 2, (_GPT - 2) % 4, do_fetch=False)
    wait_gather(rows[(_GPT - 1) % 2], slots[(_GPT - 1) % 4])
    compute(rows[(_GPT - 1) % 2], slots[(_GPT - 1) % 4])
    issue_scatter(rows[(_GPT - 1) % 2], slots[(_GPT - 1) % 4])
    wait_scatter(rows[0], slots[(_GPT - 2) % 4])
    wait_scatter(rows[1], slots[(_GPT - 1) % 4])
    plsc.subcore_barrier()
    start = 8 * (sid * 78 + jnp.minimum(sid, 2))

    @pl.when(sid < 2)
    def _read_wide():
        pltpu.sync_copy(agg_sh.at[pl.ds(start, 632)],
                        out_h.at[pl.ds(cid * _N + start, 632)])

    @pl.when(sid >= 2)
    def _read_narrow():
        pltpu.sync_copy(agg_sh.at[pl.ds(start, 624)],
                        out_h.at[pl.ds(cid * _N + start, 624)])


@functools.cache
def _get_sc_edge():
    return pl.kernel(
        _sc_edge_body,
        out_type=jax.ShapeDtypeStruct((2 * _N, _HC), jnp.float32),
        mesh=plsc.VectorSubcoreMesh(core_axis_name="c", subcore_axis_name="s",
                                    num_cores=_NC, num_subcores=_NS),
        compiler_params=pltpu.CompilerParams(needs_layout_passes=False),
        scratch_types=(
            [pltpu.VMEM((_G, _HC), jnp.float32),
             ] * 2 +
            [pltpu.VMEM((_G,), jnp.int32),
             pltpu.VMEM((_G,), jnp.int32),
             pltpu.VMEM((2 * _G,), jnp.float32),
             ] * 4 +
            [pltpu.VMEM((2 * _NP,), jnp.int32),
             pltpu.VMEM_SHARED((_NP, _HC), jnp.float32),
             ] +
            [pltpu.SemaphoreType.DMA] * 8),
    )


_BN = 2000
_BE = 4000


_BNF = 1000
_BEF = 4000
_E8P = 2 * _EPAD // 16


def _node_body(x_ref, wt_ref, am_ref, ea_ref, b_ref, xl_ref, a_ref,
               ae0_ref, ae1_ref):
    xl = jnp.dot(x_ref[...], wt_ref[...], preferred_element_type=jnp.float32)
    xl_ref[...] = xl
    a_ref[...] = jnp.dot(xl, am_ref[...], preferred_element_type=jnp.float32)
    m = jnp.dot(ea_ref[...], b_ref[...], preferred_element_type=jnp.float32)
    ae0_ref[...] = m[:, :16]
    ae1_ref[...] = m[:, 16:]


def _tc_node(x, wt, am, ea_view, bcat):
    return pl.pallas_call(
        _node_body,
        grid=(_N // _BNF,),
        in_specs=[
            pl.BlockSpec((_BNF, _HC), lambda i: (i, 0)),
            pl.BlockSpec((_HC, _HC), lambda i: (0, 0)),
            pl.BlockSpec((_HC, 4), lambda i: (0, 0)),
            pl.BlockSpec((_BEF, _HC), lambda i: (i, 0)),
            pl.BlockSpec((_HC, 32), lambda i: (0, 0)),
        ],
        out_specs=[
            pl.BlockSpec((_BNF, _HC), lambda i: (i, 0)),
            pl.BlockSpec((_BNF, 4), lambda i: (i, 0)),
            pl.BlockSpec((_BEF, 16), lambda i: (i, 0)),
            pl.BlockSpec((_BEF, 16), lambda i: (i, 0)),
        ],
        out_shape=[
            jax.ShapeDtypeStruct((_NP, _HC), jnp.float32),
            jax.ShapeDtypeStruct((_N, 4), jnp.float32),
            jax.ShapeDtypeStruct((_E8P, 16), jnp.float32),
            jax.ShapeDtypeStruct((_E8P, 16), jnp.float32),
        ],
    )(x, wt, am, ea_view, bcat)


def _combine_body(p_ref, b_ref, wt_ref, am_ref, xl_ref, a_ref):
    h = jnp.maximum(p_ref[0] + p_ref[1] + b_ref[...], 0.0)
    xl = jnp.dot(h, wt_ref[...], preferred_element_type=jnp.float32)
    xl_ref[...] = xl
    a_ref[...] = jnp.dot(xl, am_ref[...], preferred_element_type=jnp.float32)


def _tc_combine(parts, brow, wt, am):
    return pl.pallas_call(
        _combine_body,
        grid=(_N // _BN,),
        in_specs=[
            pl.BlockSpec((2, _BN, _HC), lambda i: (0, i, 0)),
            pl.BlockSpec((1, _HC), lambda i: (0, 0)),
            pl.BlockSpec((_HC, _HC), lambda i: (0, 0)),
            pl.BlockSpec((_HC, 4), lambda i: (0, 0)),
        ],
        out_specs=[
            pl.BlockSpec((_BN, _HC), lambda i: (i, 0)),
            pl.BlockSpec((_BN, 4), lambda i: (i, 0)),
        ],
        out_shape=[
            jax.ShapeDtypeStruct((_NP, _HC), jnp.float32),
            jax.ShapeDtypeStruct((_N, 4), jnp.float32),
        ],
    )(parts, brow, wt, am)


def _final_body(p_ref, b_ref, bt_ref, wo_ref, bo_ref, o_ref, sum_acc, cnt_acc):
    i = pl.program_id(0)

    @pl.when(i == 0)
    def _init():
        sum_acc[...] = jnp.zeros_like(sum_acc)
        cnt_acc[...] = jnp.zeros_like(cnt_acc)

    h = jnp.maximum(p_ref[0] + p_ref[1] + b_ref[...], 0.0)
    oh = (bt_ref[...] == lax.broadcasted_iota(jnp.int32, (1, _NG), 1)
          ).astype(jnp.float32)
    dnum = (((0,), (0,)), ((), ()))
    sum_acc[...] += lax.dot_general(oh, h, dnum,
                                    preferred_element_type=jnp.float32)
    cnt_acc[...] += lax.dot_general(oh, jnp.ones((_BN, _HC), jnp.float32),
                                    dnum, preferred_element_type=jnp.float32)

    @pl.when(i == _N // _BN - 1)
    def _fin():
        pooled = sum_acc[...] / jnp.maximum(cnt_acc[...], 1.0)
        logits = jnp.dot(pooled, wo_ref[...],
                         preferred_element_type=jnp.float32) + bo_ref[...]
        m = jnp.max(logits, axis=1, keepdims=True)
        sh = logits - m
        o_ref[...] = sh - jnp.log(jnp.sum(jnp.exp(sh), axis=1, keepdims=True))


def _tc_final(parts, brow, batch2d, wot, borow):
    return pl.pallas_call(
        _final_body,
        grid=(_N // _BN,),
        in_specs=[
            pl.BlockSpec((2, _BN, _HC), lambda i: (0, i, 0)),
            pl.BlockSpec((1, _HC), lambda i: (0, 0)),
            pl.BlockSpec((_BN, 1), lambda i: (i, 0)),
            pl.BlockSpec((_HC, _NCLS), lambda i: (0, 0)),
            pl.BlockSpec((1, _NCLS), lambda i: (0, 0)),
        ],
        out_specs=pl.BlockSpec((_NG, _NCLS), lambda i: (0, 0)),
        out_shape=jax.ShapeDtypeStruct((_NG, _NCLS), jnp.float32),
        scratch_shapes=[
            pltpu.VMEM((_NG, _HC), jnp.float32),
            pltpu.VMEM((_NG, _HC), jnp.float32),
        ],
    )(parts, brow, batch2d, wot, borow)


def _build_A(att):
    a = jnp.zeros((_HC, 4), jnp.float32)
    a = a.at[0:_C, 0].set(att[0, 0, 0:_C])
    a = a.at[_C:_HC, 1].set(att[0, 1, 0:_C])
    a = a.at[0:_C, 2].set(att[0, 0, _C:2 * _C])
    a = a.at[_C:_HC, 3].set(att[0, 1, _C:2 * _C])
    return a


def _build_B(we, att):
    ve = jnp.stack(
        [we[h * _C:(h + 1) * _C, :].T @ att[0, h, 2 * _C:] for h in range(2)],
        axis=1)
    return jnp.kron(jnp.eye(8, dtype=jnp.float32), ve)


def _pack_tbl(a):
    t = lax.bitcast_convert_type(a.astype(jnp.bfloat16).reshape(-1, 2),
                                 jnp.int32)
    return jnp.concatenate([t, jnp.zeros((2 * (_NP - _N),), jnp.int32)])


def kernel(x, edge_index, edge_attr, batch, W0, We0, att0, b0,
           W1, We1, att1, b1, Wout, bout):
    pad_e = _EPAD - _E
    src = jnp.concatenate([edge_index[0].astype(jnp.int32),
                           jnp.full((pad_e,), _N, jnp.int32)])
    dst = jnp.concatenate([edge_index[1].astype(jnp.int32),
                           jnp.full((pad_e,), _N, jnp.int32)])
    batch2d = batch.astype(jnp.int32).reshape(_N, 1)

    A0 = _build_A(att0)
    A1 = _build_A(att1)
    bcat = jnp.concatenate([_build_B(We0, att0), _build_B(We1, att1)], axis=1)
    ea_view = edge_attr.reshape(_E // 8, _HC)
    zrows = jnp.zeros((_ZROWS, _HC), jnp.float32)

    sc_edge = _get_sc_edge()
    xl0, a0, ae0p, ae1p = _tc_node(x, W0.T, A0, ea_view, bcat)
    ae0 = ae0p.reshape(-1)
    ae1 = ae1p.reshape(-1)
    parts0 = sc_edge(src, dst, _pack_tbl(a0), ae0, xl0,
                     zrows).reshape(2, _N, _HC)
    xl1, a1 = _tc_combine(parts0, b0.reshape(1, _HC), W1.T, A1)
    parts1 = sc_edge(src, dst, _pack_tbl(a1), ae1, xl1,
                     zrows).reshape(2, _N, _HC)
    return _tc_final(parts1, b1.reshape(1, _HC), batch2d,
                     Wout.T, bout.reshape(1, _NCLS))


# speedup vs baseline: 94.2814x; 1.0335x over previous
import functools

import jax
import jax.numpy as jnp
from jax import lax
from jax.experimental import pallas as pl
from jax.experimental.pallas import tpu as pltpu
from jax.experimental.pallas import tpu_sc as plsc

_N = 10000
_E = 320000
_HC = 128
_C = 64
_DE = 16
_NG = 16
_NCLS = 4

_NC = 2
_NS = 16
_NW = _NC * _NS
_G = 112
_GPT0 = 102
_GPT1 = 78
_EPAD = _NS * _G * (_GPT0 + _GPT1)
_NP = _N + 8
_ZROWS = 640


def _sc_edge_body(src_h, dst_h, tbl_h, ae_h, xl_h, z_h, out_h,
                  rows_v0, rows_v1,
                  srcq0, dstq0, aeq0, srcq1, dstq1, aeq1,
                  srcq2, dstq2, aeq2, srcq3, dstq3, aeq3,
                  tbl_v, agg_sh,
                  gsem0, gsem1, ssem0, ssem1,
                  isem0, isem1, isem2, isem3):
    cid = lax.axis_index("c")
    sid = lax.axis_index("s")

    pltpu.sync_copy(z_h, agg_sh.at[pl.ds(sid * 624, _ZROWS)])
    pltpu.sync_copy(tbl_h, tbl_v)
    plsc.subcore_barrier()

    ng = jnp.where(cid == 0, _GPT0, _GPT1)
    ebase = _G * jnp.where(cid == 0, sid * _GPT0,
                           _NS * _GPT0 + sid * _GPT1)
    rows = ((rows_v0, gsem0, ssem0), (rows_v1, gsem1, ssem1))
    slots = ((srcq0, dstq0, aeq0, isem0), (srcq1, dstq1, aeq1, isem1),
             (srcq2, dstq2, aeq2, isem2), (srcq3, dstq3, aeq3, isem3))

    def idx_copies(g, q):
        base = pl.multiple_of(ebase + g * _G, 16)
        return (pltpu.make_async_copy(src_h.at[pl.ds(base, _G)], q[0], q[3]),
                pltpu.make_async_copy(dst_h.at[pl.ds(base, _G)], q[1], q[3]),
                pltpu.make_async_copy(ae_h.at[pl.ds(base * 2, 2 * _G)],
                                      q[2], q[3]))

    def fetch_idx(g, q):
        for c in idx_copies(g, q):
            c.start()

    def wait_idx(q):
        for c in idx_copies(0, q):
            c.wait()

    def issue_gather(r, q):
        pltpu.async_copy(xl_h.at[q[0]], r[0], r[1])

    def wait_gather(r, q):
        pltpu.make_async_copy(xl_h.at[q[0]], r[0], r[1]).wait()

    def issue_scatter(r, q):
        pltpu.async_copy(r[0], agg_sh.at[q[1]], r[2], add=True)

    def wait_scatter(r, q):
        pltpu.make_async_copy(r[0], agg_sh.at[q[1]], r[2]).wait()

    def compute(r, q):
        src_v, dst_v, ae_v = q[0], q[1], q[2]
        rows_v = r[0]
        lane = lax.iota(jnp.int32, 16)

        def jbody(j, carry):
            sl = pl.ds(j * 16, 16)
            s16 = src_v[sl]
            d16 = dst_v[sl]
            wi = plsc.load_gather(tbl_v, [d16 * 2])
            wj = plsc.load_gather(tbl_v, [s16 * 2 + 1])
            ai0, ai1 = plsc.unpack(plsc.bitcast(wi, jnp.bfloat16),
                                   format=plsc.PackFormat.INTERLEAVED)
            aj0, aj1 = plsc.unpack(plsc.bitcast(wj, jnp.bfloat16),
                                   format=plsc.PackFormat.INTERLEAVED)
            ei = j * 32 + lane * 2
            ae0 = plsc.load_gather(ae_v, [ei])
            ae1 = plsc.load_gather(ae_v, [ei + 1])
            s0 = ai0 + aj0 + ae0
            s1 = ai1 + aj1 + ae1
            s0 = jnp.where(s0 >= 0.0, s0, s0 * 0.2)
            s1 = jnp.where(s1 >= 0.0, s1, s1 * 0.2)
            m = jnp.maximum(s0, s1)
            e0 = jnp.exp(s0 - m)
            e1 = jnp.exp(s1 - m)
            inv = 1.0 / (e0 + e1)
            a0v = e0 * inv
            a1v = e1 * inv
            for k in range(16):
                r = j * 16 + k
                a0s = a0v[k]
                a1s = a1v[k]
                for q in range(4):
                    rows_v[r, pl.ds(q * 16, 16)] = (
                        rows_v[r, pl.ds(q * 16, 16)] * a0s)
                for q in range(4, 8):
                    rows_v[r, pl.ds(q * 16, 16)] = (
                        rows_v[r, pl.ds(q * 16, 16)] * a1s)
            return carry

        lax.fori_loop(0, _G // 16, jbody, 0)

    def phase(g, r4, has_scatter_wait=True, do_fetch=True):
        s = r4 % 2
        wait_idx(slots[(r4 + 1) % 4])
        if has_scatter_wait:
            wait_scatter(rows[1 - s], slots[(r4 - 1) % 4])
        issue_gather(rows[1 - s], slots[(r4 + 1) % 4])
        if do_fetch:
            fetch_idx(g + 2, slots[(r4 + 2) % 4])
        wait_gather(rows[s], slots[r4])
        compute(rows[s], slots[r4])
        issue_scatter(rows[s], slots[r4])

    fetch_idx(0, slots[0])
    fetch_idx(1, slots[1])
    wait_idx(slots[0])
    issue_gather(rows[0], slots[0])
    phase(0, 0, has_scatter_wait=False)
    phase(1, 1)

    def quad(p, carry):
        for i in range(4):
            phase(2 + 4 * p + i, (2 + i) % 4)
        return carry

    lax.fori_loop(0, (ng - 6) // 4, quad, 0)
    phase(ng - 4, 2)
    phase(ng - 3, 3)
    phase(ng - 2, 0, do_fetch=False)
    wait_gather(rows[1], slots[1])
    compute(rows[1], slots[1])
    issue_scatter(rows[1], slots[1])
    wait_scatter(rows[0], slots[0])
    wait_scatter(rows[1], slots[1])
    plsc.subcore_barrier()
    start = 8 * (sid * 78 + jnp.minimum(sid, 2))

    @pl.when(sid < 2)
    def _read_wide():
        pltpu.sync_copy(agg_sh.at[pl.ds(start, 632)],
                        out_h.at[pl.ds(cid * _N + start, 632)])

    @pl.when(sid >= 2)
    def _read_narrow():
        pltpu.sync_copy(agg_sh.at[pl.ds(start, 624)],
                        out_h.at[pl.ds(cid * _N + start, 624)])


@functools.cache
def _get_sc_edge():
    return pl.kernel(
        _sc_edge_body,
        out_type=jax.ShapeDtypeStruct((2 * _N, _HC), jnp.float32),
        mesh=plsc.VectorSubcoreMesh(core_axis_name="c", subcore_axis_name="s",
                                    num_cores=_NC, num_subcores=_NS),
        compiler_params=pltpu.CompilerParams(needs_layout_passes=False),
        scratch_types=(
            [pltpu.VMEM((_G, _HC), jnp.float32),
             ] * 2 +
            [pltpu.VMEM((_G,), jnp.int32),
             pltpu.VMEM((_G,), jnp.int32),
             pltpu.VMEM((2 * _G,), jnp.float32),
             ] * 4 +
            [pltpu.VMEM((2 * _NP,), jnp.int32),
             pltpu.VMEM_SHARED((_NP, _HC), jnp.float32),
             ] +
            [pltpu.SemaphoreType.DMA] * 8),
    )


_BN = 2000
_BE = 4000


_BNF = 1000
_BEF = 4000
_E8P = 2 * _EPAD // 16


def _node_body(x_ref, wt_ref, am_ref, ea_ref, b_ref, xl_ref, a_ref,
               ae0_ref, ae1_ref):
    xl = jnp.dot(x_ref[...], wt_ref[...], preferred_element_type=jnp.float32)
    xl_ref[...] = xl
    a_ref[...] = jnp.dot(xl, am_ref[...], preferred_element_type=jnp.float32)
    m = jnp.dot(ea_ref[...], b_ref[...], preferred_element_type=jnp.float32)
    ae0_ref[...] = m[:, :16]
    ae1_ref[...] = m[:, 16:]


def _tc_node(x, wt, am, ea_view, bcat):
    return pl.pallas_call(
        _node_body,
        grid=(_N // _BNF,),
        in_specs=[
            pl.BlockSpec((_BNF, _HC), lambda i: (i, 0)),
            pl.BlockSpec((_HC, _HC), lambda i: (0, 0)),
            pl.BlockSpec((_HC, 4), lambda i: (0, 0)),
            pl.BlockSpec((_BEF, _HC), lambda i: (i, 0)),
            pl.BlockSpec((_HC, 32), lambda i: (0, 0)),
        ],
        out_specs=[
            pl.BlockSpec((_BNF, _HC), lambda i: (i, 0)),
            pl.BlockSpec((_BNF, 4), lambda i: (i, 0)),
            pl.BlockSpec((_BEF, 16), lambda i: (i, 0)),
            pl.BlockSpec((_BEF, 16), lambda i: (i, 0)),
        ],
        out_shape=[
            jax.ShapeDtypeStruct((_NP, _HC), jnp.float32),
            jax.ShapeDtypeStruct((_N, 4), jnp.float32),
            jax.ShapeDtypeStruct((_E8P, 16), jnp.float32),
            jax.ShapeDtypeStruct((_E8P, 16), jnp.float32),
        ],
    )(x, wt, am, ea_view, bcat)


def _combine_body(p_ref, b_ref, wt_ref, am_ref, xl_ref, a_ref):
    h = jnp.maximum(p_ref[0] + p_ref[1] + b_ref[...], 0.0)
    xl = jnp.dot(h, wt_ref[...], preferred_element_type=jnp.float32)
    xl_ref[...] = xl
    a_ref[...] = jnp.dot(xl, am_ref[...], preferred_element_type=jnp.float32)


def _tc_combine(parts, brow, wt, am):
    return pl.pallas_call(
        _combine_body,
        grid=(_N // _BN,),
        in_specs=[
            pl.BlockSpec((2, _BN, _HC), lambda i: (0, i, 0)),
            pl.BlockSpec((1, _HC), lambda i: (0, 0)),
            pl.BlockSpec((_HC, _HC), lambda i: (0, 0)),
            pl.BlockSpec((_HC, 4), lambda i: (0, 0)),
        ],
        out_specs=[
            pl.BlockSpec((_BN, _HC), lambda i: (i, 0)),
            pl.BlockSpec((_BN, 4), lambda i: (i, 0)),
        ],
        out_shape=[
            jax.ShapeDtypeStruct((_NP, _HC), jnp.float32),
            jax.ShapeDtypeStruct((_N, 4), jnp.float32),
        ],
    )(parts, brow, wt, am)


def _final_body(p_ref, b_ref, bt_ref, wo_ref, bo_ref, o_ref, sum_acc, cnt_acc):
    i = pl.program_id(0)

    @pl.when(i == 0)
    def _init():
        sum_acc[...] = jnp.zeros_like(sum_acc)
        cnt_acc[...] = jnp.zeros_like(cnt_acc)

    h = jnp.maximum(p_ref[0] + p_ref[1] + b_ref[...], 0.0)
    oh = (bt_ref[...] == lax.broadcasted_iota(jnp.int32, (1, _NG), 1)
          ).astype(jnp.float32)
    dnum = (((0,), (0,)), ((), ()))
    sum_acc[...] += lax.dot_general(oh, h, dnum,
                                    preferred_element_type=jnp.float32)
    cnt_acc[...] += lax.dot_general(oh, jnp.ones((_BN, _HC), jnp.float32),
                                    dnum, preferred_element_type=jnp.float32)

    @pl.when(i == _N // _BN - 1)
    def _fin():
        pooled = sum_acc[...] / jnp.maximum(cnt_acc[...], 1.0)
        logits = jnp.dot(pooled, wo_ref[...],
                         preferred_element_type=jnp.float32) + bo_ref[...]
        m = jnp.max(logits, axis=1, keepdims=True)
        sh = logits - m
        o_ref[...] = sh - jnp.log(jnp.sum(jnp.exp(sh), axis=1, keepdims=True))


def _tc_final(parts, brow, batch2d, wot, borow):
    return pl.pallas_call(
        _final_body,
        grid=(_N // _BN,),
        in_specs=[
            pl.BlockSpec((2, _BN, _HC), lambda i: (0, i, 0)),
            pl.BlockSpec((1, _HC), lambda i: (0, 0)),
            pl.BlockSpec((_BN, 1), lambda i: (i, 0)),
            pl.BlockSpec((_HC, _NCLS), lambda i: (0, 0)),
            pl.BlockSpec((1, _NCLS), lambda i: (0, 0)),
        ],
        out_specs=pl.BlockSpec((_NG, _NCLS), lambda i: (0, 0)),
        out_shape=jax.ShapeDtypeStruct((_NG, _NCLS), jnp.float32),
        scratch_shapes=[
            pltpu.VMEM((_NG, _HC), jnp.float32),
            pltpu.VMEM((_NG, _HC), jnp.float32),
        ],
    )(parts, brow, batch2d, wot, borow)


def _build_A(att):
    a = jnp.zeros((_HC, 4), jnp.float32)
    a = a.at[0:_C, 0].set(att[0, 0, 0:_C])
    a = a.at[_C:_HC, 1].set(att[0, 1, 0:_C])
    a = a.at[0:_C, 2].set(att[0, 0, _C:2 * _C])
    a = a.at[_C:_HC, 3].set(att[0, 1, _C:2 * _C])
    return a


def _build_B(we, att):
    ve = jnp.stack(
        [we[h * _C:(h + 1) * _C, :].T @ att[0, h, 2 * _C:] for h in range(2)],
        axis=1)
    return jnp.kron(jnp.eye(8, dtype=jnp.float32), ve)


def _pack_tbl(a):
    t = lax.bitcast_convert_type(a.astype(jnp.bfloat16).reshape(-1, 2),
                                 jnp.int32)
    return jnp.concatenate([t, jnp.zeros((2 * (_NP - _N),), jnp.int32)])


def kernel(x, edge_index, edge_attr, batch, W0, We0, att0, b0,
           W1, We1, att1, b1, Wout, bout):
    pad_e = _EPAD - _E
    src = jnp.concatenate([edge_index[0].astype(jnp.int32),
                           jnp.full((pad_e,), _N, jnp.int32)])
    dst = jnp.concatenate([edge_index[1].astype(jnp.int32),
                           jnp.full((pad_e,), _N, jnp.int32)])
    batch2d = batch.astype(jnp.int32).reshape(_N, 1)

    A0 = _build_A(att0)
    A1 = _build_A(att1)
    bcat = jnp.concatenate([_build_B(We0, att0), _build_B(We1, att1)], axis=1)
    ea_view = edge_attr.reshape(_E // 8, _HC)
    zrows = jnp.zeros((_ZROWS, _HC), jnp.float32)

    sc_edge = _get_sc_edge()
    xl0, a0, ae0p, ae1p = _tc_node(x, W0.T, A0, ea_view, bcat)
    ae0 = ae0p.reshape(-1)
    ae1 = ae1p.reshape(-1)
    parts0 = sc_edge(src, dst, _pack_tbl(a0), ae0, xl0,
                     zrows).reshape(2, _N, _HC)
    xl1, a1 = _tc_combine(parts0, b0.reshape(1, _HC), W1.T, A1)
    parts1 = sc_edge(src, dst, _pack_tbl(a1), ae1, xl1,
                     zrows).reshape(2, _N, _HC)
    return _tc_final(parts1, b1.reshape(1, _HC), batch2d,
                     Wout.T, bout.reshape(1, _NCLS))


# speedup vs baseline: 96.6576x; 1.0252x over previous
import functools

import jax
import jax.numpy as jnp
from jax import lax
from jax.experimental import pallas as pl
from jax.experimental.pallas import tpu as pltpu
from jax.experimental.pallas import tpu_sc as plsc

_N = 10000
_E = 320000
_HC = 128
_C = 64
_DE = 16
_NG = 16
_NCLS = 4

_NC = 2
_NS = 16
_NW = _NC * _NS
_G = 112
_GPT0 = 110
_GPT1 = 70
_EPAD = _NS * _G * (_GPT0 + _GPT1)
_NP = _N + 8
_ZROWS = 640


def _sc_edge_body(src_h, dst_h, tbl_h, ae_h, xl_h, z_h, out_h,
                  rows_v0, rows_v1,
                  srcq0, dstq0, aeq0, srcq1, dstq1, aeq1,
                  srcq2, dstq2, aeq2, srcq3, dstq3, aeq3,
                  tbl_v, agg_sh,
                  gsem0, gsem1, ssem0, ssem1,
                  isem0, isem1, isem2, isem3):
    cid = lax.axis_index("c")
    sid = lax.axis_index("s")

    pltpu.sync_copy(z_h, agg_sh.at[pl.ds(sid * 624, _ZROWS)])
    pltpu.sync_copy(tbl_h, tbl_v)
    plsc.subcore_barrier()

    ng = jnp.where(cid == 0, _GPT0, _GPT1)
    ebase = _G * jnp.where(cid == 0, sid * _GPT0,
                           _NS * _GPT0 + sid * _GPT1)
    rows = ((rows_v0, gsem0, ssem0), (rows_v1, gsem1, ssem1))
    slots = ((srcq0, dstq0, aeq0, isem0), (srcq1, dstq1, aeq1, isem1),
             (srcq2, dstq2, aeq2, isem2), (srcq3, dstq3, aeq3, isem3))

    def idx_copies(g, q):
        base = pl.multiple_of(ebase + g * _G, 16)
        return (pltpu.make_async_copy(src_h.at[pl.ds(base, _G)], q[0], q[3]),
                pltpu.make_async_copy(dst_h.at[pl.ds(base, _G)], q[1], q[3]),
                pltpu.make_async_copy(ae_h.at[pl.ds(base * 2, 2 * _G)],
                                      q[2], q[3]))

    def fetch_idx(g, q):
        for c in idx_copies(g, q):
            c.start()

    def wait_idx(q):
        for c in idx_copies(0, q):
            c.wait()

    def issue_gather(r, q):
        pltpu.async_copy(xl_h.at[q[0]], r[0], r[1])

    def wait_gather(r, q):
        pltpu.make_async_copy(xl_h.at[q[0]], r[0], r[1]).wait()

    def issue_scatter(r, q):
        pltpu.async_copy(r[0], agg_sh.at[q[1]], r[2], add=True)

    def wait_scatter(r, q):
        pltpu.make_async_copy(r[0], agg_sh.at[q[1]], r[2]).wait()

    def compute(r, q):
        src_v, dst_v, ae_v = q[0], q[1], q[2]
        rows_v = r[0]
        lane = lax.iota(jnp.int32, 16)

        def jbody(j, carry):
            sl = pl.ds(j * 16, 16)
            s16 = src_v[sl]
            d16 = dst_v[sl]
            wi = plsc.load_gather(tbl_v, [d16 * 2])
            wj = plsc.load_gather(tbl_v, [s16 * 2 + 1])
            ai0, ai1 = plsc.unpack(plsc.bitcast(wi, jnp.bfloat16),
                                   format=plsc.PackFormat.INTERLEAVED)
            aj0, aj1 = plsc.unpack(plsc.bitcast(wj, jnp.bfloat16),
                                   format=plsc.PackFormat.INTERLEAVED)
            ei = j * 32 + lane * 2
            ae0 = plsc.load_gather(ae_v, [ei])
            ae1 = plsc.load_gather(ae_v, [ei + 1])
            s0 = ai0 + aj0 + ae0
            s1 = ai1 + aj1 + ae1
            s0 = jnp.where(s0 >= 0.0, s0, s0 * 0.2)
            s1 = jnp.where(s1 >= 0.0, s1, s1 * 0.2)
            m = jnp.maximum(s0, s1)
            e0 = jnp.exp(s0 - m)
            e1 = jnp.exp(s1 - m)
            inv = 1.0 / (e0 + e1)
            a0v = e0 * inv
            a1v = e1 * inv
            for k in range(16):
                r = j * 16 + k
                a0s = a0v[k]
                a1s = a1v[k]
                for q in range(4):
                    rows_v[r, pl.ds(q * 16, 16)] = (
                        rows_v[r, pl.ds(q * 16, 16)] * a0s)
                for q in range(4, 8):
                    rows_v[r, pl.ds(q * 16, 16)] = (
                        rows_v[r, pl.ds(q * 16, 16)] * a1s)
            return carry

        lax.fori_loop(0, _G // 16, jbody, 0)

    def phase(g, r4, has_scatter_wait=True, do_fetch=True):
        s = r4 % 2
        wait_idx(slots[(r4 + 1) % 4])
        if has_scatter_wait:
            wait_scatter(rows[1 - s], slots[(r4 - 1) % 4])
        issue_gather(rows[1 - s], slots[(r4 + 1) % 4])
        if do_fetch:
            fetch_idx(g + 2, slots[(r4 + 2) % 4])
        wait_gather(rows[s], slots[r4])
        compute(rows[s], slots[r4])
        issue_scatter(rows[s], slots[r4])

    fetch_idx(0, slots[0])
    fetch_idx(1, slots[1])
    wait_idx(slots[0])
    issue_gather(rows[0], slots[0])
    phase(0, 0, has_scatter_wait=False)
    phase(1, 1)

    def quad(p, carry):
        for i in range(4):
            phase(2 + 4 * p + i, (2 + i) % 4)
        return carry

    lax.fori_loop(0, (ng - 6) // 4, quad, 0)
    phase(ng - 4, 2)
    phase(ng - 3, 3)
    phase(ng - 2, 0, do_fetch=False)
    wait_gather(rows[1], slots[1])
    compute(rows[1], slots[1])
    issue_scatter(rows[1], slots[1])
    wait_scatter(rows[0], slots[0])
    wait_scatter(rows[1], slots[1])
    plsc.subcore_barrier()
    start = 8 * (sid * 78 + jnp.minimum(sid, 2))

    @pl.when(sid < 2)
    def _read_wide():
        pltpu.sync_copy(agg_sh.at[pl.ds(start, 632)],
                        out_h.at[pl.ds(cid * _N + start, 632)])

    @pl.when(sid >= 2)
    def _read_narrow():
        pltpu.sync_copy(agg_sh.at[pl.ds(start, 624)],
                        out_h.at[pl.ds(cid * _N + start, 624)])


@functools.cache
def _get_sc_edge():
    return pl.kernel(
        _sc_edge_body,
        out_type=jax.ShapeDtypeStruct((2 * _N, _HC), jnp.float32),
        mesh=plsc.VectorSubcoreMesh(core_axis_name="c", subcore_axis_name="s",
                                    num_cores=_NC, num_subcores=_NS),
        compiler_params=pltpu.CompilerParams(needs_layout_passes=False),
        scratch_types=(
            [pltpu.VMEM((_G, _HC), jnp.float32),
             ] * 2 +
            [pltpu.VMEM((_G,), jnp.int32),
             pltpu.VMEM((_G,), jnp.int32),
             pltpu.VMEM((2 * _G,), jnp.float32),
             ] * 4 +
            [pltpu.VMEM((2 * _NP,), jnp.int32),
             pltpu.VMEM_SHARED((_NP, _HC), jnp.float32),
             ] +
            [pltpu.SemaphoreType.DMA] * 8),
    )


_BN = 2000
_BE = 4000


_BNF = 1000
_BEF = 4000
_E8P = 2 * _EPAD // 16


def _node_body(x_ref, wt_ref, am_ref, ea_ref, b_ref, xl_ref, a_ref,
               ae0_ref, ae1_ref):
    xl = jnp.dot(x_ref[...], wt_ref[...], preferred_element_type=jnp.float32)
    xl_ref[...] = xl
    a_ref[...] = jnp.dot(xl, am_ref[...], preferred_element_type=jnp.float32)
    m = jnp.dot(ea_ref[...], b_ref[...], preferred_element_type=jnp.float32)
    ae0_ref[...] = m[:, :16]
    ae1_ref[...] = m[:, 16:]


def _tc_node(x, wt, am, ea_view, bcat):
    return pl.pallas_call(
        _node_body,
        grid=(_N // _BNF,),
        in_specs=[
            pl.BlockSpec((_BNF, _HC), lambda i: (i, 0)),
            pl.BlockSpec((_HC, _HC), lambda i: (0, 0)),
            pl.BlockSpec((_HC, 4), lambda i: (0, 0)),
            pl.BlockSpec((_BEF, _HC), lambda i: (i, 0)),
            pl.BlockSpec((_HC, 32), lambda i: (0, 0)),
        ],
        out_specs=[
            pl.BlockSpec((_BNF, _HC), lambda i: (i, 0)),
            pl.BlockSpec((_BNF, 4), lambda i: (i, 0)),
            pl.BlockSpec((_BEF, 16), lambda i: (i, 0)),
            pl.BlockSpec((_BEF, 16), lambda i: (i, 0)),
        ],
        out_shape=[
            jax.ShapeDtypeStruct((_NP, _HC), jnp.float32),
            jax.ShapeDtypeStruct((_N, 4), jnp.float32),
            jax.ShapeDtypeStruct((_E8P, 16), jnp.float32),
            jax.ShapeDtypeStruct((_E8P, 16), jnp.float32),
        ],
    )(x, wt, am, ea_view, bcat)


def _combine_body(p_ref, b_ref, wt_ref, am_ref, xl_ref, a_ref):
    h = jnp.maximum(p_ref[0] + p_ref[1] + b_ref[...], 0.0)
    xl = jnp.dot(h, wt_ref[...], preferred_element_type=jnp.float32)
    xl_ref[...] = xl
    a_ref[...] = jnp.dot(xl, am_ref[...], preferred_element_type=jnp.float32)


def _tc_combine(parts, brow, wt, am):
    return pl.pallas_call(
        _combine_body,
        grid=(_N // _BN,),
        in_specs=[
            pl.BlockSpec((2, _BN, _HC), lambda i: (0, i, 0)),
            pl.BlockSpec((1, _HC), lambda i: (0, 0)),
            pl.BlockSpec((_HC, _HC), lambda i: (0, 0)),
            pl.BlockSpec((_HC, 4), lambda i: (0, 0)),
        ],
        out_specs=[
            pl.BlockSpec((_BN, _HC), lambda i: (i, 0)),
            pl.BlockSpec((_BN, 4), lambda i: (i, 0)),
        ],
        out_shape=[
            jax.ShapeDtypeStruct((_NP, _HC), jnp.float32),
            jax.ShapeDtypeStruct((_N, 4), jnp.float32),
        ],
    )(parts, brow, wt, am)


def _final_body(p_ref, b_ref, bt_ref, wo_ref, bo_ref, o_ref, sum_acc, cnt_acc):
    i = pl.program_id(0)

    @pl.when(i == 0)
    def _init():
        sum_acc[...] = jnp.zeros_like(sum_acc)
        cnt_acc[...] = jnp.zeros_like(cnt_acc)

    h = jnp.maximum(p_ref[0] + p_ref[1] + b_ref[...], 0.0)
    oh = (bt_ref[...] == lax.broadcasted_iota(jnp.int32, (1, _NG), 1)
          ).astype(jnp.float32)
    dnum = (((0,), (0,)), ((), ()))
    sum_acc[...] += lax.dot_general(oh, h, dnum,
                                    preferred_element_type=jnp.float32)
    cnt_acc[...] += lax.dot_general(oh, jnp.ones((_BN, _HC), jnp.float32),
                                    dnum, preferred_element_type=jnp.float32)

    @pl.when(i == _N // _BN - 1)
    def _fin():
        pooled = sum_acc[...] / jnp.maximum(cnt_acc[...], 1.0)
        logits = jnp.dot(pooled, wo_ref[...],
                         preferred_element_type=jnp.float32) + bo_ref[...]
        m = jnp.max(logits, axis=1, keepdims=True)
        sh = logits - m
        o_ref[...] = sh - jnp.log(jnp.sum(jnp.exp(sh), axis=1, keepdims=True))


def _tc_final(parts, brow, batch2d, wot, borow):
    return pl.pallas_call(
        _final_body,
        grid=(_N // _BN,),
        in_specs=[
            pl.BlockSpec((2, _BN, _HC), lambda i: (0, i, 0)),
            pl.BlockSpec((1, _HC), lambda i: (0, 0)),
            pl.BlockSpec((_BN, 1), lambda i: (i, 0)),
            pl.BlockSpec((_HC, _NCLS), lambda i: (0, 0)),
            pl.BlockSpec((1, _NCLS), lambda i: (0, 0)),
        ],
        out_specs=pl.BlockSpec((_NG, _NCLS), lambda i: (0, 0)),
        out_shape=jax.ShapeDtypeStruct((_NG, _NCLS), jnp.float32),
        scratch_shapes=[
            pltpu.VMEM((_NG, _HC), jnp.float32),
            pltpu.VMEM((_NG, _HC), jnp.float32),
        ],
    )(parts, brow, batch2d, wot, borow)


def _build_A(att):
    a = jnp.zeros((_HC, 4), jnp.float32)
    a = a.at[0:_C, 0].set(att[0, 0, 0:_C])
    a = a.at[_C:_HC, 1].set(att[0, 1, 0:_C])
    a = a.at[0:_C, 2].set(att[0, 0, _C:2 * _C])
    a = a.at[_C:_HC, 3].set(att[0, 1, _C:2 * _C])
    return a


def _build_B(we, att):
    ve = jnp.stack(
        [we[h * _C:(h + 1) * _C, :].T @ att[0, h, 2 * _C:] for h in range(2)],
        axis=1)
    return jnp.kron(jnp.eye(8, dtype=jnp.float32), ve)


def _pack_tbl(a):
    t = lax.bitcast_convert_type(a.astype(jnp.bfloat16).reshape(-1, 2),
                                 jnp.int32)
    return jnp.concatenate([t, jnp.zeros((2 * (_NP - _N),), jnp.int32)])


def kernel(x, edge_index, edge_attr, batch, W0, We0, att0, b0,
           W1, We1, att1, b1, Wout, bout):
    pad_e = _EPAD - _E
    src = jnp.concatenate([edge_index[0].astype(jnp.int32),
                           jnp.full((pad_e,), _N, jnp.int32)])
    dst = jnp.concatenate([edge_index[1].astype(jnp.int32),
                           jnp.full((pad_e,), _N, jnp.int32)])
    batch2d = batch.astype(jnp.int32).reshape(_N, 1)

    A0 = _build_A(att0)
    A1 = _build_A(att1)
    bcat = jnp.concatenate([_build_B(We0, att0), _build_B(We1, att1)], axis=1)
    ea_view = edge_attr.reshape(_E // 8, _HC)
    zrows = jnp.zeros((_ZROWS, _HC), jnp.float32)

    sc_edge = _get_sc_edge()
    xl0, a0, ae0p, ae1p = _tc_node(x, W0.T, A0, ea_view, bcat)
    ae0 = ae0p.reshape(-1)
    ae1 = ae1p.reshape(-1)
    parts0 = sc_edge(src, dst, _pack_tbl(a0), ae0, xl0,
                     zrows).reshape(2, _N, _HC)
    xl1, a1 = _tc_combine(parts0, b0.reshape(1, _HC), W1.T, A1)
    parts1 = sc_edge(src, dst, _pack_tbl(a1), ae1, xl1,
                     zrows).reshape(2, _N, _HC)
    return _tc_final(parts1, b1.reshape(1, _HC), batch2d,
                     Wout.T, bout.reshape(1, _NCLS))


# speedup vs baseline: 96.8117x; 1.0016x over previous
import functools

import jax
import jax.numpy as jnp
from jax import lax
from jax.experimental import pallas as pl
from jax.experimental.pallas import tpu as pltpu
from jax.experimental.pallas import tpu_sc as plsc

_N = 10000
_E = 320000
_HC = 128
_C = 64
_DE = 16
_NG = 16
_NCLS = 4

_NC = 2
_NS = 16
_NW = _NC * _NS
_G = 112
_GPT0 = 110
_GPT1 = 70
_EPAD = _NS * _G * (_GPT0 + _GPT1)
_NP = _N + 8
_ZROWS = 640


def _sc_edge_body(src_h, dst_h, tbl_h, ae_h, xl_h, z_h, out_h,
                  rows_v0, rows_v1,
                  srcq0, dstq0, aeq0, srcq1, dstq1, aeq1,
                  srcq2, dstq2, aeq2, srcq3, dstq3, aeq3,
                  tbl_v, agg_sh,
                  gsem0, gsem1, ssem0, ssem1,
                  isem0, isem1, isem2, isem3):
    cid = lax.axis_index("c")
    sid = lax.axis_index("s")

    pltpu.sync_copy(z_h, agg_sh.at[pl.ds(sid * 624, _ZROWS)])
    pltpu.sync_copy(tbl_h, tbl_v)
    plsc.subcore_barrier()

    ng = jnp.where(cid == 0, _GPT0, _GPT1)
    ebase = _G * jnp.where(cid == 0, sid * _GPT0,
                           _NS * _GPT0 + sid * _GPT1)
    rows = ((rows_v0, gsem0, ssem0), (rows_v1, gsem1, ssem1))
    slots = ((srcq0, dstq0, aeq0, isem0), (srcq1, dstq1, aeq1, isem1),
             (srcq2, dstq2, aeq2, isem2), (srcq3, dstq3, aeq3, isem3))

    def idx_copies(g, q):
        base = pl.multiple_of(ebase + g * _G, 16)
        return (pltpu.make_async_copy(src_h.at[pl.ds(base, _G)], q[0], q[3]),
                pltpu.make_async_copy(dst_h.at[pl.ds(base, _G)], q[1], q[3]),
                pltpu.make_async_copy(ae_h.at[pl.ds(base * 2, 2 * _G)],
                                      q[2], q[3]))

    def fetch_idx(g, q):
        for c in idx_copies(g, q):
            c.start()

    def wait_idx(q):
        for c in idx_copies(0, q):
            c.wait()

    def issue_gather(r, q):
        pltpu.async_copy(xl_h.at[q[0]], r[0], r[1])

    def wait_gather(r, q):
        pltpu.make_async_copy(xl_h.at[q[0]], r[0], r[1]).wait()

    def issue_scatter(r, q):
        pltpu.async_copy(r[0], agg_sh.at[q[1]], r[2], add=True)

    def wait_scatter(r, q):
        pltpu.make_async_copy(r[0], agg_sh.at[q[1]], r[2]).wait()

    def compute(r, q):
        src_v, dst_v, ae_v = q[0], q[1], q[2]
        rows_v = r[0]
        lane = lax.iota(jnp.int32, 16)

        def jbody(j, carry):
            sl = pl.ds(j * 16, 16)
            s16 = src_v[sl]
            d16 = dst_v[sl]
            wi = plsc.load_gather(tbl_v, [d16 * 2])
            wj = plsc.load_gather(tbl_v, [s16 * 2 + 1])
            ai0, ai1 = plsc.unpack(plsc.bitcast(wi, jnp.bfloat16),
                                   format=plsc.PackFormat.INTERLEAVED)
            aj0, aj1 = plsc.unpack(plsc.bitcast(wj, jnp.bfloat16),
                                   format=plsc.PackFormat.INTERLEAVED)
            ei = j * 32 + lane * 2
            ae0 = plsc.load_gather(ae_v, [ei])
            ae1 = plsc.load_gather(ae_v, [ei + 1])
            s0 = ai0 + aj0 + ae0
            s1 = ai1 + aj1 + ae1
            s0 = jnp.where(s0 >= 0.0, s0, s0 * 0.2)
            s1 = jnp.where(s1 >= 0.0, s1, s1 * 0.2)
            m = jnp.maximum(s0, s1)
            e0 = jnp.exp(s0 - m)
            e1 = jnp.exp(s1 - m)
            inv = 1.0 / (e0 + e1)
            a0v = e0 * inv
            a1v = e1 * inv
            for k in range(16):
                r = j * 16 + k
                a0s = a0v[k]
                a1s = a1v[k]
                for q in range(4):
                    rows_v[r, pl.ds(q * 16, 16)] = (
                        rows_v[r, pl.ds(q * 16, 16)] * a0s)
                for q in range(4, 8):
                    rows_v[r, pl.ds(q * 16, 16)] = (
                        rows_v[r, pl.ds(q * 16, 16)] * a1s)
            return carry

        lax.fori_loop(0, _G // 16, jbody, 0)

    def phase(g, r4, has_scatter_wait=True, do_fetch=True):
        s = r4 % 2
        wait_idx(slots[(r4 + 1) % 4])
        if has_scatter_wait:
            wait_scatter(rows[1 - s], slots[(r4 - 1) % 4])
        issue_gather(rows[1 - s], slots[(r4 + 1) % 4])
        if do_fetch:
            fetch_idx(g + 2, slots[(r4 + 2) % 4])
        wait_gather(rows[s], slots[r4])
        compute(rows[s], slots[r4])
        issue_scatter(rows[s], slots[r4])

    fetch_idx(0, slots[0])
    fetch_idx(1, slots[1])
    wait_idx(slots[0])
    issue_gather(rows[0], slots[0])
    phase(0, 0, has_scatter_wait=False)
    phase(1, 1)

    def quad(p, carry):
        for i in range(4):
            phase(2 + 4 * p + i, (2 + i) % 4)
        return carry

    lax.fori_loop(0, (ng - 6) // 4, quad, 0)
    phase(ng - 4, 2)
    phase(ng - 3, 3)
    phase(ng - 2, 0, do_fetch=False)
    wait_gather(rows[1], slots[1])
    compute(rows[1], slots[1])
    issue_scatter(rows[1], slots[1])
    wait_scatter(rows[0], slots[0])
    wait_scatter(rows[1], slots[1])
    plsc.subcore_barrier()
    start = 8 * (sid * 78 + jnp.minimum(sid, 2))

    @pl.when(sid < 2)
    def _read_wide():
        pltpu.sync_copy(agg_sh.at[pl.ds(start, 632)],
                        out_h.at[pl.ds(cid * _N + start, 632)])

    @pl.when(sid >= 2)
    def _read_narrow():
        pltpu.sync_copy(agg_sh.at[pl.ds(start, 624)],
                        out_h.at[pl.ds(cid * _N + start, 624)])


@functools.cache
def _get_sc_edge():
    return pl.kernel(
        _sc_edge_body,
        out_type=jax.ShapeDtypeStruct((2 * _N, _HC), jnp.float32),
        mesh=plsc.VectorSubcoreMesh(core_axis_name="c", subcore_axis_name="s",
                                    num_cores=_NC, num_subcores=_NS),
        compiler_params=pltpu.CompilerParams(needs_layout_passes=False),
        scratch_types=(
            [pltpu.VMEM((_G, _HC), jnp.float32),
             ] * 2 +
            [pltpu.VMEM((_G,), jnp.int32),
             pltpu.VMEM((_G,), jnp.int32),
             pltpu.VMEM((2 * _G,), jnp.float32),
             ] * 4 +
            [pltpu.VMEM((2 * _NP,), jnp.int32),
             pltpu.VMEM_SHARED((_NP, _HC), jnp.float32),
             ] +
            [pltpu.SemaphoreType.DMA] * 8),
    )


_BN = 2000


_BNF = 1000
_BEF = 4000
_E8P = 2 * _EPAD // 16


def _node_body(x_ref, wt_ref, am_ref, ea_ref, b_ref, xl_ref, a_ref,
               ae0_ref, ae1_ref):
    xl = jnp.dot(x_ref[...], wt_ref[...], preferred_element_type=jnp.float32)
    xl_ref[...] = xl
    a_ref[...] = jnp.dot(xl, am_ref[...], preferred_element_type=jnp.float32)
    m = jnp.dot(ea_ref[...], b_ref[...], preferred_element_type=jnp.float32)
    ae0_ref[...] = m[:, :16]
    ae1_ref[...] = m[:, 16:]


def _tc_node(x, wt, am, ea_view, bcat):
    return pl.pallas_call(
        _node_body,
        grid=(_N // _BNF,),
        in_specs=[
            pl.BlockSpec((_BNF, _HC), lambda i: (i, 0)),
            pl.BlockSpec((_HC, _HC), lambda i: (0, 0)),
            pl.BlockSpec((_HC, 4), lambda i: (0, 0)),
            pl.BlockSpec((_BEF, _HC), lambda i: (i, 0)),
            pl.BlockSpec((_HC, 32), lambda i: (0, 0)),
        ],
        out_specs=[
            pl.BlockSpec((_BNF, _HC), lambda i: (i, 0)),
            pl.BlockSpec((_BNF, 4), lambda i: (i, 0)),
            pl.BlockSpec((_BEF, 16), lambda i: (i, 0)),
            pl.BlockSpec((_BEF, 16), lambda i: (i, 0)),
        ],
        out_shape=[
            jax.ShapeDtypeStruct((_NP, _HC), jnp.float32),
            jax.ShapeDtypeStruct((_N, 4), jnp.float32),
            jax.ShapeDtypeStruct((_E8P, 16), jnp.float32),
            jax.ShapeDtypeStruct((_E8P, 16), jnp.float32),
        ],
    )(x, wt, am, ea_view, bcat)


def _combine_body(p_ref, b_ref, wt_ref, am_ref, xl_ref, a_ref):
    h = jnp.maximum(p_ref[0] + p_ref[1] + b_ref[...], 0.0)
    xl = jnp.dot(h, wt_ref[...], preferred_element_type=jnp.float32)
    xl_ref[...] = xl
    a_ref[...] = jnp.dot(xl, am_ref[...], preferred_element_type=jnp.float32)


def _tc_combine(parts, brow, wt, am):
    return pl.pallas_call(
        _combine_body,
        grid=(_N // _BN,),
        in_specs=[
            pl.BlockSpec((2, _BN, _HC), lambda i: (0, i, 0)),
            pl.BlockSpec((1, _HC), lambda i: (0, 0)),
            pl.BlockSpec((_HC, _HC), lambda i: (0, 0)),
            pl.BlockSpec((_HC, 4), lambda i: (0, 0)),
        ],
        out_specs=[
            pl.BlockSpec((_BN, _HC), lambda i: (i, 0)),
            pl.BlockSpec((_BN, 4), lambda i: (i, 0)),
        ],
        out_shape=[
            jax.ShapeDtypeStruct((_NP, _HC), jnp.float32),
            jax.ShapeDtypeStruct((_N, 4), jnp.float32),
        ],
    )(parts, brow, wt, am)


def _final_body(p_ref, b_ref, bt_ref, wo_ref, bo_ref, o_ref, sum_acc, cnt_acc):
    i = pl.program_id(0)

    @pl.when(i == 0)
    def _init():
        sum_acc[...] = jnp.zeros_like(sum_acc)
        cnt_acc[...] = jnp.zeros_like(cnt_acc)

    h = jnp.maximum(p_ref[0] + p_ref[1] + b_ref[...], 0.0)
    oh = (bt_ref[...] == lax.broadcasted_iota(jnp.int32, (1, _NG), 1)
          ).astype(jnp.float32)
    dnum = (((0,), (0,)), ((), ()))
    sum_acc[...] += lax.dot_general(oh, h, dnum,
                                    preferred_element_type=jnp.float32)
    cnt_acc[...] += lax.dot_general(oh, jnp.ones((_BN, _HC), jnp.float32),
                                    dnum, preferred_element_type=jnp.float32)

    @pl.when(i == _N // _BN - 1)
    def _fin():
        pooled = sum_acc[...] / jnp.maximum(cnt_acc[...], 1.0)
        logits = jnp.dot(pooled, wo_ref[...],
                         preferred_element_type=jnp.float32) + bo_ref[...]
        m = jnp.max(logits, axis=1, keepdims=True)
        sh = logits - m
        o_ref[...] = sh - jnp.log(jnp.sum(jnp.exp(sh), axis=1, keepdims=True))


def _tc_final(parts, brow, batch2d, wot, borow):
    return pl.pallas_call(
        _final_body,
        grid=(_N // _BN,),
        in_specs=[
            pl.BlockSpec((2, _BN, _HC), lambda i: (0, i, 0)),
            pl.BlockSpec((1, _HC), lambda i: (0, 0)),
            pl.BlockSpec((_BN, 1), lambda i: (i, 0)),
            pl.BlockSpec((_HC, _NCLS), lambda i: (0, 0)),
            pl.BlockSpec((1, _NCLS), lambda i: (0, 0)),
        ],
        out_specs=pl.BlockSpec((_NG, _NCLS), lambda i: (0, 0)),
        out_shape=jax.ShapeDtypeStruct((_NG, _NCLS), jnp.float32),
        scratch_shapes=[
            pltpu.VMEM((_NG, _HC), jnp.float32),
            pltpu.VMEM((_NG, _HC), jnp.float32),
        ],
    )(parts, brow, batch2d, wot, borow)


def _build_A(att):
    a = jnp.zeros((_HC, 4), jnp.float32)
    a = a.at[0:_C, 0].set(att[0, 0, 0:_C])
    a = a.at[_C:_HC, 1].set(att[0, 1, 0:_C])
    a = a.at[0:_C, 2].set(att[0, 0, _C:2 * _C])
    a = a.at[_C:_HC, 3].set(att[0, 1, _C:2 * _C])
    return a


def _build_B(we, att):
    ve = jnp.stack(
        [we[h * _C:(h + 1) * _C, :].T @ att[0, h, 2 * _C:] for h in range(2)],
        axis=1)
    return jnp.kron(jnp.eye(8, dtype=jnp.float32), ve)


def _pack_tbl(a):
    t = lax.bitcast_convert_type(a.astype(jnp.bfloat16).reshape(-1, 2),
                                 jnp.int32)
    return jnp.concatenate([t, jnp.zeros((2 * (_NP - _N),), jnp.int32)])


def kernel(x, edge_index, edge_attr, batch, W0, We0, att0, b0,
           W1, We1, att1, b1, Wout, bout):
    pad_e = _EPAD - _E
    src = jnp.concatenate([edge_index[0].astype(jnp.int32),
                           jnp.full((pad_e,), _N, jnp.int32)])
    dst = jnp.concatenate([edge_index[1].astype(jnp.int32),
                           jnp.full((pad_e,), _N, jnp.int32)])
    batch2d = batch.astype(jnp.int32).reshape(_N, 1)

    A0 = _build_A(att0)
    A1 = _build_A(att1)
    bcat = jnp.concatenate([_build_B(We0, att0), _build_B(We1, att1)], axis=1)
    ea_view = edge_attr.reshape(_E // 8, _HC)
    zrows = jnp.zeros((_ZROWS, _HC), jnp.float32)

    sc_edge = _get_sc_edge()
    xl0, a0, ae0p, ae1p = _tc_node(x, W0.T, A0, ea_view, bcat)
    ae0 = ae0p.reshape(-1)
    ae1 = ae1p.reshape(-1)
    parts0 = sc_edge(src, dst, _pack_tbl(a0), ae0, xl0,
                     zrows).reshape(2, _N, _HC)
    xl1, a1 = _tc_combine(parts0, b0.reshape(1, _HC), W1.T, A1)
    parts1 = sc_edge(src, dst, _pack_tbl(a1), ae1, xl1,
                     zrows).reshape(2, _N, _HC)
    return _tc_final(parts1, b1.reshape(1, _HC), batch2d,
                     Wout.T, bout.reshape(1, _NCLS))
